# baseline (speedup 1.0000x reference)
import functools

import jax
import jax.numpy as jnp
from jax import lax
from jax.experimental import pallas as pl
from jax.experimental.pallas import tpu as pltpu
from jax.experimental.pallas import tpu_sc as plsc

_BS, _H, _W = 8, 512, 512
_L = _H * _W
_ALPHA = 0.1
_POS_SEED = 5.0
_NEG_SEED = -5.0

_CHUNK = 8192
_CPM = _L // _CHUNK
_ROWS = _CHUNK // 128
_WINR = _ROWS + 2
_EXT = _CHUNK + 2 * 128
_DUMP = 128
_AROWS = (_L + _EXT + _DUMP) // 128
_ASTRIDE = _AROWS * 128


def _dilate_body(t_ref, o_ref):
    t2 = t_ref[0]
    h = t2
    for s in (1, 2, 3):
        zc = jnp.zeros((_H, s), jnp.float32)
        h = jnp.maximum(h, jnp.concatenate([t2[:, s:], zc], axis=1))
        h = jnp.maximum(h, jnp.concatenate([zc, t2[:, : _W - s]], axis=1))
    v = h
    for s in (1, 2, 3):
        zr = jnp.zeros((s, _W), jnp.float32)
        v = jnp.maximum(v, jnp.concatenate([h[s:, :], zr], axis=0))
        v = jnp.maximum(v, jnp.concatenate([zr, h[: _H - s, :]], axis=0))
    o_ref[0] = (v > 0.0).astype(jnp.float32)


def _dilate(t3):
    return pl.pallas_call(
        _dilate_body,
        grid=(_BS,),
        in_specs=[pl.BlockSpec((1, _H, _W), lambda m: (m, 0, 0))],
        out_specs=pl.BlockSpec((1, _H, _W), lambda m: (m, 0, 0)),
        out_shape=jax.ShapeDtypeStruct((_BS, _H, _W), jnp.float32),
    )(t3)


_NSC = 2
_SUBC = 16
_MAPS_PER_CORE = _BS // _NSC
_CHK = _L // _SUBC
_SUB = 8192
_NSUB = _CHK // _SUB
_VPS = _SUB // 16
_VROWS = _CHK // 128 + 2
_EXTR = (_EXT + 127) // 128
_EXTSLOTS = (_EXTR + _SUBC - 1) // _SUBC


def _sc_body(x_hbm, t_hbm, aug_hbm, a1_hbm, a2_hbm, cnt_hbm,
             xb, tb, gb, vpos, vfp, vneg, idxb, grow, srow, trow,
             crow, tbl, csp, sem):
    c = lax.axis_index("c")
    s = lax.axis_index("s")
    lane = lax.iota(jnp.int32, 16)

    def popcnt(mask):
        return plsc.cumsum(jnp.where(mask, 1, 0))[15]

    def scatter_local(valref, cnt, base, dump, ahbm):
        nrows = (cnt + 127) // 128

        def mkrow(j, _):
            for v in range(8):
                p = j * 128 + v * 16 + lane
                iv = jnp.where(p < cnt, base + p, dump + (p & 63))
                idxb[j, pl.ds(v * 16, 16)] = iv
            return 0

        lax.fori_loop(0, nrows, mkrow, 0)

        def fire(j, _):
            pltpu.async_copy(valref.at[pl.ds(j * 128, 128)],
                             ahbm.at[idxb.at[j]], sem)
            return 0

        lax.fori_loop(0, nrows, fire, 0)

        def drain(j, _):
            pltpu.make_async_copy(valref.at[pl.ds(0, 128)],
                                  ahbm.at[idxb.at[0]], sem).wait()
            return 0

        lax.fori_loop(0, nrows, drain, 0)

    def extend(ahbm, n, base):
        def eloop(jj, _):
            row = s + jj * _SUBC

            @pl.when(row < _EXTR)
            def _():
                for v in range(8):
                    p = n + row * 128 + v * 16 + lane
                    grow[0, pl.ds(v * 16, 16)] = base + lax.rem(p, n)
                    srow[0, pl.ds(v * 16, 16)] = base + p
                pltpu.sync_copy(ahbm.at[grow.at[0]], trow.at[0])
                pltpu.sync_copy(trow.at[0], ahbm.at[srow.at[0]])
            return 0

        lax.fori_loop(0, _EXTSLOTS, eloop, 0)

    def seed(ahbm, n, base, dump, value):
        @pl.when(jnp.logical_and(s == 0, n == 0))
        def _():
            for v in range(8):
                p = v * 16 + lane
                trow[0, pl.ds(v * 16, 16)] = jnp.full((16,), value,
                                                      jnp.float32)
                idxb[0, pl.ds(v * 16, 16)] = jnp.where(
                    p == 0, base, dump + (p & 63))
            pltpu.sync_copy(trow.at[0], ahbm.at[idxb.at[0]])

    def per_map(mi, _unused):
        m = c * _MAPS_PER_CORE + mi
        gbase = m * _L + s * _CHK
        abase = m * _ASTRIDE
        dump = abase + _L + _EXT

        def sub_loop(sub, carry):
            off = gbase + sub * _SUB
            pltpu.sync_copy(x_hbm.at[pl.ds(off, _SUB)], xb)
            pltpu.sync_copy(t_hbm.at[pl.ds(off, _SUB)], tb)
            pltpu.sync_copy(aug_hbm.at[pl.ds(off, _SUB)], gb)

            def vloop(v, carry2):
                w1, w2, w3 = carry2
                o = v * 16
                xv = xb[pl.ds(o, 16)]
                tv = tb[pl.ds(o, 16)]
                gv = gb[pl.ds(o, 16)]
                pos = tv > 0.0
                neg = gv == 0.0
                fp = jnp.logical_and(xv > 0.0, neg)
                packed = (jnp.where(pos, 1, 0) + jnp.where(fp, 1 << 10, 0)
                          + jnp.where(neg, 1 << 20, 0))
                pk = plsc.cumsum(packed)[15]
                plsc.store_compressed(vpos.at[pl.ds(w1, 16)], xv, mask=pos)
                plsc.store_compressed(vfp.at[pl.ds(w2, 16)], xv, mask=fp)
                plsc.store_compressed(vneg.at[pl.ds(w3, 16)], xv, mask=neg)
                return (w1 + (pk & 0x3FF), w2 + ((pk >> 10) & 0x3FF),
                        w3 + (pk >> 20))

            return lax.fori_loop(0, _VPS, vloop, carry, unroll=8)

        z0 = jnp.int32(0)
        wp1, wp2, wp3 = lax.fori_loop(0, _NSUB, sub_loop, (z0, z0, z0))

        crow[pl.ds(0, 16)] = jnp.where(
            lane == 0, wp1, jnp.where(lane == 1, wp2,
                                      jnp.where(lane == 2, wp3, 0)))
        pltpu.sync_copy(crow, csp.at[c, s])
        plsc.subcore_barrier()
        pltpu.sync_copy(csp.at[c], tbl)

        def offs(j, carry):
            o1, o2, o3, t1, t2, t3 = carry
            rv = tbl[j, pl.ds(0, 16)]
            v1 = rv[0]
            v2 = rv[1]
            v3 = rv[2]
            before = (j < s).astype(jnp.int32)
            return (o1 + before * v1, o2 + before * v2, o3 + before * v3,
                    t1 + v1, t2 + v2, t3 + v3)

        z = jnp.int32(0)
        o1, o2, o3, n1, nf, nn = lax.fori_loop(0, _SUBC, offs,
                                               (z, z, z, z, z, z))
        use_fp = nf > 0
        n2 = jnp.where(use_fp, nf, nn)
        o2c = jnp.where(use_fp, o2, o3)
        c2c = jnp.where(use_fp, wp2, wp3)

        scatter_local(vpos, wp1, abase + o1, dump, a1_hbm)

        @pl.when(use_fp)
        def _():
            scatter_local(vfp, c2c, abase + o2c, dump, a2_hbm)

        @pl.when(jnp.logical_not(use_fp))
        def _():
            scatter_local(vneg, c2c, abase + o2c, dump, a2_hbm)

        seed(a1_hbm, n1, abase, dump, _POS_SEED)
        seed(a2_hbm, n2, abase, dump, _NEG_SEED)
        n1e = jnp.maximum(n1, 1)
        n2e = jnp.maximum(n2, 1)
        plsc.subcore_barrier()

        extend(a1_hbm, n1e, abase)
        extend(a2_hbm, n2e, abase)

        @pl.when(s == 0)
        def _():
            crow[pl.ds(0, 16)] = jnp.where(
                lane == 0, n1e, jnp.where(lane == 1, n2e, 0))
            pltpu.sync_copy(crow, cnt_hbm.at[pl.ds(m * 16, 16)])

        plsc.subcore_barrier()
        return 0

    lax.fori_loop(0, _MAPS_PER_CORE, per_map, 0)


def _compact_sc(xf, tf, augf):
    mesh = plsc.VectorSubcoreMesh(core_axis_name="c", subcore_axis_name="s")
    a1, a2, cnt = pl.kernel(
        _sc_body,
        out_type=[
            jax.ShapeDtypeStruct((_BS * _ASTRIDE,), jnp.float32),
            jax.ShapeDtypeStruct((_BS * _ASTRIDE,), jnp.float32),
            jax.ShapeDtypeStruct((_BS * 16,), jnp.int32),
        ],
        mesh=mesh,
        compiler_params=pltpu.CompilerParams(needs_layout_passes=False),
        scratch_types=[
            pltpu.VMEM((_SUB,), jnp.float32),
            pltpu.VMEM((_SUB,), jnp.float32),
            pltpu.VMEM((_SUB,), jnp.float32),
            pltpu.VMEM((_VROWS * 128,), jnp.float32),
            pltpu.VMEM((_VROWS * 128,), jnp.float32),
            pltpu.VMEM((_VROWS * 128,), jnp.float32),
            pltpu.VMEM((_VROWS, 128), jnp.int32),
            pltpu.VMEM((1, 128), jnp.int32),
            pltpu.VMEM((1, 128), jnp.int32),
            pltpu.VMEM((1, 128), jnp.float32),
            pltpu.VMEM((16,), jnp.int32),
            pltpu.VMEM((_SUBC, 16), jnp.int32),
            pltpu.VMEM_SHARED((_NSC, _SUBC, 16), jnp.int32),
            pltpu.SemaphoreType.DMA,
        ],
    )(xf.reshape(-1), tf.reshape(-1), augf.reshape(-1))
    return (a1.reshape(_BS, _AROWS, 128), a2.reshape(_BS, _AROWS, 128),
            cnt.reshape(_BS, 16))


def _compact_jnp(xf, tf, augf):
    ar = jnp.arange(_L)

    def one(x, t, aug):
        pos = t > 0.0
        neg = aug == 0.0
        fp = jnp.logical_and(x > 0.0, neg)
        nfp = jnp.sum(fp.astype(jnp.int32))
        use_fp = nfp > 0
        cho = jnp.where(use_fp, fp, neg)

        def compact(mask, seed):
            n = jnp.sum(mask.astype(jnp.int32))
            key = jnp.where(mask, ar, ar + _L)
            vals = x[jnp.argsort(key)]
            vals = jnp.where(n == 0, vals.at[0].set(seed), vals)
            neff = jnp.maximum(n, 1)
            idx = jnp.arange(_ASTRIDE) % neff
            return vals[idx], neff

        a1, n1 = compact(pos, _POS_SEED)
        a2, n2 = compact(cho, _NEG_SEED)
        return a1, a2, n1, n2

    a1, a2, n1, n2 = jax.vmap(one)(xf, tf, augf)
    counts = jnp.zeros((_BS, 16), jnp.int32)
    counts = counts.at[:, 0].set(n1).at[:, 1].set(n2)
    return (a1.reshape(_BS, _AROWS, 128), a2.reshape(_BS, _AROWS, 128),
            counts)


def _loss_body(counts_ref, x_ref, t_ref, a1_hbm, a2_hbm, o_ref,
               w1, w2, sem1, sem2):
    i = pl.program_id(0)
    m = i // _CPM
    k = i - m * _CPM
    n1 = counts_ref[m, 0]
    n2 = counts_ref[m, 1]
    start = k * _CHUNK

    s1 = lax.rem(start, n1)
    s2 = lax.rem(start, n2)
    r1 = lax.rem(s1, 128)
    r2 = lax.rem(s2, 128)
    row1 = s1 // 128
    row2 = s2 // 128

    c1 = pltpu.make_async_copy(a1_hbm.at[m, pl.ds(row1, _WINR)], w1, sem1)
    c2 = pltpu.make_async_copy(a2_hbm.at[m, pl.ds(row2, _WINR)], w2, sem2)
    c1.start()
    c2.start()
    c1.wait()
    c2.wait()

    lane = lax.broadcasted_iota(jnp.int32, (_ROWS, 128), 1)

    def unshift(w_ref, r):
        wv = w_ref[...]
        u = pltpu.roll(wv, lax.rem(128 - r, 128), axis=1)
        return jnp.where(lane < 128 - r, u[0:_ROWS], u[1:_ROWS + 1])

    g1 = unshift(w1, r1)
    g2 = unshift(w2, r2)

    x = x_ref[0]
    t = t_ref[0]
    z1 = g1 * x
    f1 = jnp.maximum(z1, 0.0) - z1 * t + jnp.log1p(jnp.exp(-jnp.abs(z1)))
    f2 = jnp.maximum(-g1, 0.0) + jnp.log1p(jnp.exp(-jnp.abs(g1)))
    z3 = g1 * g2
    f3 = jnp.maximum(z3, 0.0) + jnp.log1p(jnp.exp(-jnp.abs(z3)))
    f = (f1 + f2 + _ALPHA * f3) * (1.0 / (_BS * _L))
    acc = jnp.zeros((8, 128), jnp.float32)
    for rr in range(0, _ROWS, 8):
        acc = acc + f[rr:rr + 8]

    @pl.when(i == 0)
    def _init():
        o_ref[...] = jnp.zeros((8, 128), jnp.float32)

    o_ref[...] += acc


def _loss(counts, x3, t3, a1, a2):
    nprog = _BS * _CPM
    return pl.pallas_call(
        _loss_body,
        grid=(nprog,),
        in_specs=[
            pl.BlockSpec(memory_space=pltpu.SMEM),
            pl.BlockSpec((1, _ROWS, 128), lambda i: (i // _CPM, i % _CPM, 0)),
            pl.BlockSpec((1, _ROWS, 128), lambda i: (i // _CPM, i % _CPM, 0)),
            pl.BlockSpec(memory_space=pltpu.HBM),
            pl.BlockSpec(memory_space=pltpu.HBM),
        ],
        out_specs=pl.BlockSpec((8, 128), lambda i: (0, 0)),
        out_shape=jax.ShapeDtypeStruct((8, 128), jnp.float32),
        scratch_shapes=[
            pltpu.VMEM((_WINR, 128), jnp.float32),
            pltpu.VMEM((_WINR, 128), jnp.float32),
            pltpu.SemaphoreType.DMA,
            pltpu.SemaphoreType.DMA,
        ],
    )(counts, x3, t3, a1, a2)


def kernel(input, target):
    x3 = input.reshape(_BS, _CPM * _ROWS, 128)
    t3 = target.reshape(_BS, _CPM * _ROWS, 128)
    aug = _dilate(target.reshape(_BS, _H, _W))

    xf = input.reshape(_BS, _L)
    tf = target.reshape(_BS, _L)
    augf = aug.reshape(_BS, _L)
    a1, a2, counts = _compact_sc(xf, tf, augf)

    partials = _loss(counts, x3, t3, a1, a2)
    return jnp.sum(partials).reshape(())


# speedup vs baseline: 4.1251x; 4.1251x over previous
import functools

import jax
import jax.numpy as jnp
from jax import lax
from jax.experimental import pallas as pl
from jax.experimental.pallas import tpu as pltpu
from jax.experimental.pallas import tpu_sc as plsc

_BS, _H, _W = 8, 512, 512
_L = _H * _W
_ALPHA = 0.1
_POS_SEED = 5.0
_NEG_SEED = -5.0

_CHUNK = 8192
_CPM = _L // _CHUNK
_ROWS = _CHUNK // 128
_WINR = _ROWS + 2
_EXT = _CHUNK + 2 * 128
_WB = 2048
_WBMAX = (_L + _EXT + _WB - 1) // _WB
_ASTRIDE = _WBMAX * _WB
_AROWS = _ASTRIDE // 128


def _dilate_body(t_ref, o_ref):
    t2 = t_ref[0]
    h = t2
    for s in (1, 2, 3):
        zc = jnp.zeros((_H, s), jnp.float32)
        h = jnp.maximum(h, jnp.concatenate([t2[:, s:], zc], axis=1))
        h = jnp.maximum(h, jnp.concatenate([zc, t2[:, : _W - s]], axis=1))
    v = h
    for s in (1, 2, 3):
        zr = jnp.zeros((s, _W), jnp.float32)
        v = jnp.maximum(v, jnp.concatenate([h[s:, :], zr], axis=0))
        v = jnp.maximum(v, jnp.concatenate([zr, h[: _H - s, :]], axis=0))
    o_ref[0] = (v > 0.0).astype(jnp.float32)


def _dilate(t3):
    return pl.pallas_call(
        _dilate_body,
        grid=(_BS,),
        in_specs=[pl.BlockSpec((1, _H, _W), lambda m: (m, 0, 0))],
        out_specs=pl.BlockSpec((1, _H, _W), lambda m: (m, 0, 0)),
        out_shape=jax.ShapeDtypeStruct((_BS, _H, _W), jnp.float32),
    )(t3)


_NSC = 2
_SUBC = 16
_MAPS_PER_CORE = _BS // _NSC
_CHK = _L // _SUBC
_SUB = 8192
_NSUB = _CHK // _SUB
_VPS = _SUB // 16
_VROWS = _CHK // 128 + 2
_EXTR = (_EXT + 127) // 128
_EXTSLOTS = (_EXTR + _SUBC - 1) // _SUBC


def _sc_body(x_hbm, t_hbm, aug_hbm, a1_hbm, a2_hbm, cnt_hbm,
             xb, tb, gb, vpos, vfp, vneg, idxb, grow, srow, trow,
             crow, tbl, csp, a1sp, a2sp, sem):
    c = lax.axis_index("c")
    s = lax.axis_index("s")
    lane = lax.iota(jnp.int32, 16)
    dump = _L + _EXT

    def popcnt(mask):
        return plsc.cumsum(jnp.where(mask, 1, 0))[15]

    def scatter_local(valref, cnt, base, asp):
        nrows = (cnt + 127) // 128

        def mkrow(j, _):
            for v in range(8):
                p = j * 128 + v * 16 + lane
                iv = jnp.where(p < cnt, base + p, dump + (p & 63))
                idxb[j, pl.ds(v * 16, 16)] = iv
            return 0

        lax.fori_loop(0, nrows, mkrow, 0)

        def fire(j, _):
            pltpu.async_copy(valref.at[pl.ds(j * 128, 128)],
                             asp.at[idxb.at[j]], sem)
            return 0

        lax.fori_loop(0, nrows, fire, 0)

        def drain(j, _):
            pltpu.make_async_copy(valref.at[pl.ds(0, 128)],
                                  asp.at[idxb.at[0]], sem).wait()
            return 0

        lax.fori_loop(0, nrows, drain, 0)

    def extend(asp, n):
        def eloop(jj, _):
            row = s + jj * _SUBC

            @pl.when(row < _EXTR)
            def _():
                for v in range(8):
                    p = n + row * 128 + v * 16 + lane
                    grow[0, pl.ds(v * 16, 16)] = lax.rem(p, n)
                    srow[0, pl.ds(v * 16, 16)] = p
                pltpu.sync_copy(asp.at[grow.at[0]], trow.at[0])
                pltpu.sync_copy(trow.at[0], asp.at[srow.at[0]])
            return 0

        lax.fori_loop(0, _EXTSLOTS, eloop, 0)

    def seed(asp, n, value):
        @pl.when(jnp.logical_and(s == 0, n == 0))
        def _():
            for v in range(8):
                p = v * 16 + lane
                trow[0, pl.ds(v * 16, 16)] = jnp.full((16,), value,
                                                      jnp.float32)
                idxb[0, pl.ds(v * 16, 16)] = jnp.where(
                    p == 0, 0, dump + (p & 63))
            pltpu.sync_copy(trow.at[0], asp.at[idxb.at[0]])

    def writeback(asp, n, ahbm, abase):
        trips = (n + _EXT + _WB - 1) // _WB

        def wloop(jj, _):
            ch = s + jj * _SUBC

            @pl.when(ch < trips)
            def _():
                pltpu.sync_copy(asp.at[pl.ds(ch * _WB, _WB)],
                                ahbm.at[pl.ds(abase + ch * _WB, _WB)])
            return 0

        lax.fori_loop(0, (_WBMAX + _SUBC - 1) // _SUBC, wloop, 0)

    def per_map(mi, _unused):
        m = c * _MAPS_PER_CORE + mi
        gbase = m * _L + s * _CHK
        abase = m * _ASTRIDE

        def sub_loop(sub, carry):
            off = gbase + sub * _SUB
            pltpu.sync_copy(x_hbm.at[pl.ds(off, _SUB)], xb)
            pltpu.sync_copy(t_hbm.at[pl.ds(off, _SUB)], tb)
            pltpu.sync_copy(aug_hbm.at[pl.ds(off, _SUB)], gb)

            def vloop(v, carry2):
                w1, w2, w3 = carry2
                o = v * 16
                xv = xb[pl.ds(o, 16)]
                tv = tb[pl.ds(o, 16)]
                gv = gb[pl.ds(o, 16)]
                pos = tv > 0.0
                neg = gv == 0.0
                fp = jnp.logical_and(xv > 0.0, neg)
                packed = (jnp.where(pos, 1, 0) + jnp.where(fp, 1 << 10, 0)
                          + jnp.where(neg, 1 << 20, 0))
                pk = plsc.cumsum(packed)[15]
                plsc.store_compressed(vpos.at[pl.ds(w1, 16)], xv, mask=pos)
                plsc.store_compressed(vfp.at[pl.ds(w2, 16)], xv, mask=fp)
                plsc.store_compressed(vneg.at[pl.ds(w3, 16)], xv, mask=neg)
                return (w1 + (pk & 0x3FF), w2 + ((pk >> 10) & 0x3FF),
                        w3 + (pk >> 20))

            return lax.fori_loop(0, _VPS, vloop, carry, unroll=8)

        z0 = jnp.int32(0)
        wp1, wp2, wp3 = lax.fori_loop(0, _NSUB, sub_loop, (z0, z0, z0))

        crow[pl.ds(0, 16)] = jnp.where(
            lane == 0, wp1, jnp.where(lane == 1, wp2,
                                      jnp.where(lane == 2, wp3, 0)))
        pltpu.sync_copy(crow, csp.at[c, s])
        plsc.subcore_barrier()
        pltpu.sync_copy(csp.at[c], tbl)

        def offs(j, carry):
            o1, o2, o3, t1, t2, t3 = carry
            rv = tbl[j, pl.ds(0, 16)]
            v1 = rv[0]
            v2 = rv[1]
            v3 = rv[2]
            before = (j < s).astype(jnp.int32)
            return (o1 + before * v1, o2 + before * v2, o3 + before * v3,
                    t1 + v1, t2 + v2, t3 + v3)

        z = jnp.int32(0)
        o1, o2, o3, n1, nf, nn = lax.fori_loop(0, _SUBC, offs,
                                               (z, z, z, z, z, z))
        use_fp = nf > 0
        n2 = jnp.where(use_fp, nf, nn)
        o2c = jnp.where(use_fp, o2, o3)
        c2c = jnp.where(use_fp, wp2, wp3)

        scatter_local(vpos, wp1, o1, a1sp)

        @pl.when(use_fp)
        def _():
            scatter_local(vfp, c2c, o2c, a2sp)

        @pl.when(jnp.logical_not(use_fp))
        def _():
            scatter_local(vneg, c2c, o2c, a2sp)

        seed(a1sp, n1, _POS_SEED)
        seed(a2sp, n2, _NEG_SEED)
        n1e = jnp.maximum(n1, 1)
        n2e = jnp.maximum(n2, 1)
        plsc.subcore_barrier()

        extend(a1sp, n1e)
        extend(a2sp, n2e)
        plsc.subcore_barrier()

        writeback(a1sp, n1e, a1_hbm, abase)
        writeback(a2sp, n2e, a2_hbm, abase)

        @pl.when(s == 0)
        def _():
            crow[pl.ds(0, 16)] = jnp.where(
                lane == 0, n1e, jnp.where(lane == 1, n2e, 0))
            pltpu.sync_copy(crow, cnt_hbm.at[pl.ds(m * 16, 16)])

        plsc.subcore_barrier()
        return 0

    lax.fori_loop(0, _MAPS_PER_CORE, per_map, 0)


def _compact_sc(xf, tf, augf):
    mesh = plsc.VectorSubcoreMesh(core_axis_name="c", subcore_axis_name="s")
    a1, a2, cnt = pl.kernel(
        _sc_body,
        out_type=[
            jax.ShapeDtypeStruct((_BS * _ASTRIDE,), jnp.float32),
            jax.ShapeDtypeStruct((_BS * _ASTRIDE,), jnp.float32),
            jax.ShapeDtypeStruct((_BS * 16,), jnp.int32),
        ],
        mesh=mesh,
        compiler_params=pltpu.CompilerParams(needs_layout_passes=False),
        scratch_types=[
            pltpu.VMEM((_SUB,), jnp.float32),
            pltpu.VMEM((_SUB,), jnp.float32),
            pltpu.VMEM((_SUB,), jnp.float32),
            pltpu.VMEM((_VROWS * 128,), jnp.float32),
            pltpu.VMEM((_VROWS * 128,), jnp.float32),
            pltpu.VMEM((_VROWS * 128,), jnp.float32),
            pltpu.VMEM((_VROWS, 128), jnp.int32),
            pltpu.VMEM((1, 128), jnp.int32),
            pltpu.VMEM((1, 128), jnp.int32),
            pltpu.VMEM((1, 128), jnp.float32),
            pltpu.VMEM((16,), jnp.int32),
            pltpu.VMEM((_SUBC, 16), jnp.int32),
            pltpu.VMEM_SHARED((_NSC, _SUBC, 16), jnp.int32),
            pltpu.VMEM_SHARED((_ASTRIDE,), jnp.float32),
            pltpu.VMEM_SHARED((_ASTRIDE,), jnp.float32),
            pltpu.SemaphoreType.DMA,
        ],
    )(xf.reshape(-1), tf.reshape(-1), augf.reshape(-1))
    return (a1.reshape(_BS, _AROWS, 128), a2.reshape(_BS, _AROWS, 128),
            cnt.reshape(_BS, 16))


def _compact_jnp(xf, tf, augf):
    ar = jnp.arange(_L)

    def one(x, t, aug):
        pos = t > 0.0
        neg = aug == 0.0
        fp = jnp.logical_and(x > 0.0, neg)
        nfp = jnp.sum(fp.astype(jnp.int32))
        use_fp = nfp > 0
        cho = jnp.where(use_fp, fp, neg)

        def compact(mask, seed):
            n = jnp.sum(mask.astype(jnp.int32))
            key = jnp.where(mask, ar, ar + _L)
            vals = x[jnp.argsort(key)]
            vals = jnp.where(n == 0, vals.at[0].set(seed), vals)
            neff = jnp.maximum(n, 1)
            idx = jnp.arange(_ASTRIDE) % neff
            return vals[idx], neff

        a1, n1 = compact(pos, _POS_SEED)
        a2, n2 = compact(cho, _NEG_SEED)
        return a1, a2, n1, n2

    a1, a2, n1, n2 = jax.vmap(one)(xf, tf, augf)
    counts = jnp.zeros((_BS, 16), jnp.int32)
    counts = counts.at[:, 0].set(n1).at[:, 1].set(n2)
    return (a1.reshape(_BS, _AROWS, 128), a2.reshape(_BS, _AROWS, 128),
            counts)


def _loss_body(counts_ref, x_ref, t_ref, a1_hbm, a2_hbm, o_ref,
               w1, w2, sem1, sem2):
    i = pl.program_id(0)
    m = i // _CPM
    k = i - m * _CPM
    n1 = counts_ref[m, 0]
    n2 = counts_ref[m, 1]
    start = k * _CHUNK

    s1 = lax.rem(start, n1)
    s2 = lax.rem(start, n2)
    r1 = lax.rem(s1, 128)
    r2 = lax.rem(s2, 128)
    row1 = s1 // 128
    row2 = s2 // 128

    c1 = pltpu.make_async_copy(a1_hbm.at[m, pl.ds(row1, _WINR)], w1, sem1)
    c2 = pltpu.make_async_copy(a2_hbm.at[m, pl.ds(row2, _WINR)], w2, sem2)
    c1.start()
    c2.start()
    c1.wait()
    c2.wait()

    lane = lax.broadcasted_iota(jnp.int32, (_ROWS, 128), 1)

    def unshift(w_ref, r):
        wv = w_ref[...]
        u = pltpu.roll(wv, lax.rem(128 - r, 128), axis=1)
        return jnp.where(lane < 128 - r, u[0:_ROWS], u[1:_ROWS + 1])

    g1 = unshift(w1, r1)
    g2 = unshift(w2, r2)

    x = x_ref[0]
    t = t_ref[0]
    z1 = g1 * x
    f1 = jnp.maximum(z1, 0.0) - z1 * t + jnp.log1p(jnp.exp(-jnp.abs(z1)))
    f2 = jnp.maximum(-g1, 0.0) + jnp.log1p(jnp.exp(-jnp.abs(g1)))
    z3 = g1 * g2
    f3 = jnp.maximum(z3, 0.0) + jnp.log1p(jnp.exp(-jnp.abs(z3)))
    f = (f1 + f2 + _ALPHA * f3) * (1.0 / (_BS * _L))
    acc = jnp.zeros((8, 128), jnp.float32)
    for rr in range(0, _ROWS, 8):
        acc = acc + f[rr:rr + 8]

    @pl.when(i == 0)
    def _init():
        o_ref[...] = jnp.zeros((8, 128), jnp.float32)

    o_ref[...] += acc


def _loss(counts, x3, t3, a1, a2):
    nprog = _BS * _CPM
    return pl.pallas_call(
        _loss_body,
        grid=(nprog,),
        in_specs=[
            pl.BlockSpec(memory_space=pltpu.SMEM),
            pl.BlockSpec((1, _ROWS, 128), lambda i: (i // _CPM, i % _CPM, 0)),
            pl.BlockSpec((1, _ROWS, 128), lambda i: (i // _CPM, i % _CPM, 0)),
            pl.BlockSpec(memory_space=pltpu.HBM),
            pl.BlockSpec(memory_space=pltpu.HBM),
        ],
        out_specs=pl.BlockSpec((8, 128), lambda i: (0, 0)),
        out_shape=jax.ShapeDtypeStruct((8, 128), jnp.float32),
        scratch_shapes=[
            pltpu.VMEM((_WINR, 128), jnp.float32),
            pltpu.VMEM((_WINR, 128), jnp.float32),
            pltpu.SemaphoreType.DMA,
            pltpu.SemaphoreType.DMA,
        ],
    )(counts, x3, t3, a1, a2)


def kernel(input, target):
    x3 = input.reshape(_BS, _CPM * _ROWS, 128)
    t3 = target.reshape(_BS, _CPM * _ROWS, 128)
    aug = _dilate(target.reshape(_BS, _H, _W))

    xf = input.reshape(_BS, _L)
    tf = target.reshape(_BS, _L)
    augf = aug.reshape(_BS, _L)
    a1, a2, counts = _compact_sc(xf, tf, augf)

    partials = _loss(counts, x3, t3, a1, a2)
    return jnp.sum(partials).reshape(())


# speedup vs baseline: 4.1270x; 1.0005x over previous
import functools

import jax
import jax.numpy as jnp
from jax import lax
from jax.experimental import pallas as pl
from jax.experimental.pallas import tpu as pltpu
from jax.experimental.pallas import tpu_sc as plsc

_BS, _H, _W = 8, 512, 512
_L = _H * _W
_ALPHA = 0.1
_POS_SEED = 5.0
_NEG_SEED = -5.0

_CHUNK = 8192
_CPM = _L // _CHUNK
_ROWS = _CHUNK // 128
_WINR = _ROWS + 2
_EXT = _CHUNK + 2 * 128
_WB = 2048
_WBMAX = (_L + _EXT + _WB - 1) // _WB
_ASTRIDE = _WBMAX * _WB
_AROWS = _ASTRIDE // 128


def _dilate_body(t_ref, o_ref):
    t2 = t_ref[0]
    h = t2
    for s in (1, 2, 3):
        zc = jnp.zeros((_H, s), jnp.float32)
        h = jnp.maximum(h, jnp.concatenate([t2[:, s:], zc], axis=1))
        h = jnp.maximum(h, jnp.concatenate([zc, t2[:, : _W - s]], axis=1))
    v = h
    for s in (1, 2, 3):
        zr = jnp.zeros((s, _W), jnp.float32)
        v = jnp.maximum(v, jnp.concatenate([h[s:, :], zr], axis=0))
        v = jnp.maximum(v, jnp.concatenate([zr, h[: _H - s, :]], axis=0))
    o_ref[0] = (v > 0.0).astype(jnp.float32)


def _dilate(t3):
    return pl.pallas_call(
        _dilate_body,
        grid=(_BS,),
        in_specs=[pl.BlockSpec((1, _H, _W), lambda m: (m, 0, 0))],
        out_specs=pl.BlockSpec((1, _H, _W), lambda m: (m, 0, 0)),
        out_shape=jax.ShapeDtypeStruct((_BS, _H, _W), jnp.float32),
    )(t3)


_NSC = 2
_SUBC = 16
_MAPS_PER_CORE = _BS // _NSC
_CHK = _L // _SUBC
_SUB = 8192
_NSUB = _CHK // _SUB
_VPS = _SUB // 16
_VROWS = _CHK // 128 + 2
_EXTR = (_EXT + 127) // 128
_EXTSLOTS = (_EXTR + _SUBC - 1) // _SUBC


def _sc_body(x_hbm, t_hbm, aug_hbm, a1_hbm, a2_hbm, cnt_hbm,
             xb, tb, gb, vpos, vfp, vneg, idxb, grow, srow, trow,
             crow, tbl, csp, a1sp, a2sp, sem):
    c = lax.axis_index("c")
    s = lax.axis_index("s")
    lane = lax.iota(jnp.int32, 16)
    dump = _L + _EXT

    def popcnt(mask):
        return plsc.cumsum(jnp.where(mask, 1, 0))[15]

    def scatter_local(valref, cnt, base, asp):
        nrows = (cnt + 127) // 128

        def mkrow(j, _):
            for v in range(8):
                p = j * 128 + v * 16 + lane
                iv = jnp.where(p < cnt, base + p, dump + (p & 63))
                idxb[j, pl.ds(v * 16, 16)] = iv
            return 0

        lax.fori_loop(0, nrows, mkrow, 0)

        def fire(j, _):
            pltpu.async_copy(valref.at[pl.ds(j * 128, 128)],
                             asp.at[idxb.at[j]], sem)
            return 0

        lax.fori_loop(0, nrows, fire, 0)

        def drain(j, _):
            pltpu.make_async_copy(valref.at[pl.ds(0, 128)],
                                  asp.at[idxb.at[0]], sem).wait()
            return 0

        lax.fori_loop(0, nrows, drain, 0)

    def extend(asp, n):
        def eloop(jj, _):
            row = s + jj * _SUBC

            @pl.when(row < _EXTR)
            def _():
                for v in range(8):
                    p = n + row * 128 + v * 16 + lane
                    grow[0, pl.ds(v * 16, 16)] = lax.rem(p, n)
                    srow[0, pl.ds(v * 16, 16)] = p
                pltpu.sync_copy(asp.at[grow.at[0]], trow.at[0])
                pltpu.sync_copy(trow.at[0], asp.at[srow.at[0]])
            return 0

        lax.fori_loop(0, _EXTSLOTS, eloop, 0)

    def seed(asp, n, value):
        @pl.when(jnp.logical_and(s == 0, n == 0))
        def _():
            for v in range(8):
                p = v * 16 + lane
                trow[0, pl.ds(v * 16, 16)] = jnp.full((16,), value,
                                                      jnp.float32)
                idxb[0, pl.ds(v * 16, 16)] = jnp.where(
                    p == 0, 0, dump + (p & 63))
            pltpu.sync_copy(trow.at[0], asp.at[idxb.at[0]])

    def writeback(asp, n, ahbm, abase):
        trips = (n + _EXT + _WB - 1) // _WB

        def wloop(jj, _):
            ch = s + jj * _SUBC

            @pl.when(ch < trips)
            def _():
                pltpu.sync_copy(asp.at[pl.ds(ch * _WB, _WB)],
                                ahbm.at[pl.ds(abase + ch * _WB, _WB)])
            return 0

        lax.fori_loop(0, (_WBMAX + _SUBC - 1) // _SUBC, wloop, 0)

    def per_map(mi, _unused):
        m = c * _MAPS_PER_CORE + mi
        gbase = m * _L + s * _CHK
        abase = m * _ASTRIDE

        def sub_loop(sub, carry):
            off = gbase + sub * _SUB
            pltpu.sync_copy(x_hbm.at[pl.ds(off, _SUB)], xb)
            pltpu.sync_copy(t_hbm.at[pl.ds(off, _SUB)], tb)
            pltpu.sync_copy(aug_hbm.at[pl.ds(off, _SUB)], gb)

            def vloop(v, carry2):
                w1, w2, w3 = carry2
                o = v * 16
                xv = xb[pl.ds(o, 16)]
                tv = tb[pl.ds(o, 16)]
                gv = gb[pl.ds(o, 16)]
                pos = tv > 0.0
                neg = gv == 0.0
                fp = jnp.logical_and(xv > 0.0, neg)
                packed = (jnp.where(pos, 1, 0) + jnp.where(fp, 1 << 10, 0)
                          + jnp.where(neg, 1 << 20, 0))
                pk = plsc.cumsum(packed)[15]
                plsc.store_compressed(vpos.at[pl.ds(w1, 16)], xv, mask=pos)
                plsc.store_compressed(vfp.at[pl.ds(w2, 16)], xv, mask=fp)
                plsc.store_compressed(vneg.at[pl.ds(w3, 16)], xv, mask=neg)
                return (w1 + (pk & 0x3FF), w2 + ((pk >> 10) & 0x3FF),
                        w3 + (pk >> 20))

            return lax.fori_loop(0, _VPS, vloop, carry, unroll=8)

        z0 = jnp.int32(0)
        wp1, wp2, wp3 = lax.fori_loop(0, _NSUB, sub_loop, (z0, z0, z0))

        crow[pl.ds(0, 16)] = jnp.where(
            lane == 0, wp1, jnp.where(lane == 1, wp2,
                                      jnp.where(lane == 2, wp3, 0)))
        pltpu.sync_copy(crow, csp.at[c, s])
        plsc.subcore_barrier()
        pltpu.sync_copy(csp.at[c], tbl)

        def offs(j, carry):
            o1, o2, o3, t1, t2, t3 = carry
            rv = tbl[j, pl.ds(0, 16)]
            v1 = rv[0]
            v2 = rv[1]
            v3 = rv[2]
            before = (j < s).astype(jnp.int32)
            return (o1 + before * v1, o2 + before * v2, o3 + before * v3,
                    t1 + v1, t2 + v2, t3 + v3)

        z = jnp.int32(0)
        o1, o2, o3, n1, nf, nn = lax.fori_loop(0, _SUBC, offs,
                                               (z, z, z, z, z, z))
        use_fp = nf > 0
        n2 = jnp.where(use_fp, nf, nn)
        o2c = jnp.where(use_fp, o2, o3)
        c2c = jnp.where(use_fp, wp2, wp3)

        scatter_local(vpos, wp1, o1, a1sp)

        @pl.when(use_fp)
        def _():
            scatter_local(vfp, c2c, o2c, a2sp)

        @pl.when(jnp.logical_not(use_fp))
        def _():
            scatter_local(vneg, c2c, o2c, a2sp)

        seed(a1sp, n1, _POS_SEED)
        seed(a2sp, n2, _NEG_SEED)
        n1e = jnp.maximum(n1, 1)
        n2e = jnp.maximum(n2, 1)
        plsc.subcore_barrier()

        extend(a1sp, n1e)
        extend(a2sp, n2e)
        plsc.subcore_barrier()

        writeback(a1sp, n1e, a1_hbm, abase)
        writeback(a2sp, n2e, a2_hbm, abase)

        @pl.when(s == 0)
        def _():
            crow[pl.ds(0, 16)] = jnp.where(
                lane == 0, n1e, jnp.where(lane == 1, n2e, 0))
            pltpu.sync_copy(crow, cnt_hbm.at[pl.ds(m * 16, 16)])

        plsc.subcore_barrier()
        return 0

    lax.fori_loop(0, _MAPS_PER_CORE, per_map, 0)


def _compact_sc(xf, tf, augf):
    mesh = plsc.VectorSubcoreMesh(core_axis_name="c", subcore_axis_name="s")
    a1, a2, cnt = pl.kernel(
        _sc_body,
        out_type=[
            jax.ShapeDtypeStruct((_BS * _ASTRIDE,), jnp.float32),
            jax.ShapeDtypeStruct((_BS * _ASTRIDE,), jnp.float32),
            jax.ShapeDtypeStruct((_BS * 16,), jnp.int32),
        ],
        mesh=mesh,
        compiler_params=pltpu.CompilerParams(needs_layout_passes=False,
                                             use_tc_tiling_on_sc=True),
        scratch_types=[
            pltpu.VMEM((_SUB,), jnp.float32),
            pltpu.VMEM((_SUB,), jnp.float32),
            pltpu.VMEM((_SUB,), jnp.float32),
            pltpu.VMEM((_VROWS * 128,), jnp.float32),
            pltpu.VMEM((_VROWS * 128,), jnp.float32),
            pltpu.VMEM((_VROWS * 128,), jnp.float32),
            pltpu.VMEM((_VROWS, 128), jnp.int32),
            pltpu.VMEM((1, 128), jnp.int32),
            pltpu.VMEM((1, 128), jnp.int32),
            pltpu.VMEM((1, 128), jnp.float32),
            pltpu.VMEM((16,), jnp.int32),
            pltpu.VMEM((_SUBC, 16), jnp.int32),
            pltpu.VMEM_SHARED((_NSC, _SUBC, 16), jnp.int32),
            pltpu.VMEM_SHARED((_ASTRIDE,), jnp.float32),
            pltpu.VMEM_SHARED((_ASTRIDE,), jnp.float32),
            pltpu.SemaphoreType.DMA,
        ],
    )(xf.reshape(-1), tf.reshape(-1), augf.reshape(-1))
    return (a1.reshape(_BS, _AROWS, 128), a2.reshape(_BS, _AROWS, 128),
            cnt.reshape(_BS, 16))


def _compact_jnp(xf, tf, augf):
    ar = jnp.arange(_L)

    def one(x, t, aug):
        pos = t > 0.0
        neg = aug == 0.0
        fp = jnp.logical_and(x > 0.0, neg)
        nfp = jnp.sum(fp.astype(jnp.int32))
        use_fp = nfp > 0
        cho = jnp.where(use_fp, fp, neg)

        def compact(mask, seed):
            n = jnp.sum(mask.astype(jnp.int32))
            key = jnp.where(mask, ar, ar + _L)
            vals = x[jnp.argsort(key)]
            vals = jnp.where(n == 0, vals.at[0].set(seed), vals)
            neff = jnp.maximum(n, 1)
            idx = jnp.arange(_ASTRIDE) % neff
            return vals[idx], neff

        a1, n1 = compact(pos, _POS_SEED)
        a2, n2 = compact(cho, _NEG_SEED)
        return a1, a2, n1, n2

    a1, a2, n1, n2 = jax.vmap(one)(xf, tf, augf)
    counts = jnp.zeros((_BS, 16), jnp.int32)
    counts = counts.at[:, 0].set(n1).at[:, 1].set(n2)
    return (a1.reshape(_BS, _AROWS, 128), a2.reshape(_BS, _AROWS, 128),
            counts)


def _loss_body(counts_ref, x_ref, t_ref, a1_hbm, a2_hbm, o_ref,
               w1, w2, sem1, sem2):
    i = pl.program_id(0)
    m = i // _CPM
    k = i - m * _CPM
    n1 = counts_ref[m, 0]
    n2 = counts_ref[m, 1]
    start = k * _CHUNK

    s1 = lax.rem(start, n1)
    s2 = lax.rem(start, n2)
    r1 = lax.rem(s1, 128)
    r2 = lax.rem(s2, 128)
    row1 = s1 // 128
    row2 = s2 // 128

    c1 = pltpu.make_async_copy(a1_hbm.at[m, pl.ds(row1, _WINR)], w1, sem1)
    c2 = pltpu.make_async_copy(a2_hbm.at[m, pl.ds(row2, _WINR)], w2, sem2)
    c1.start()
    c2.start()
    c1.wait()
    c2.wait()

    lane = lax.broadcasted_iota(jnp.int32, (_ROWS, 128), 1)

    def unshift(w_ref, r):
        wv = w_ref[...]
        u = pltpu.roll(wv, lax.rem(128 - r, 128), axis=1)
        return jnp.where(lane < 128 - r, u[0:_ROWS], u[1:_ROWS + 1])

    g1 = unshift(w1, r1)
    g2 = unshift(w2, r2)

    x = x_ref[0]
    t = t_ref[0]
    z1 = g1 * x
    f1 = jnp.maximum(z1, 0.0) - z1 * t + jnp.log1p(jnp.exp(-jnp.abs(z1)))
    f2 = jnp.maximum(-g1, 0.0) + jnp.log1p(jnp.exp(-jnp.abs(g1)))
    z3 = g1 * g2
    f3 = jnp.maximum(z3, 0.0) + jnp.log1p(jnp.exp(-jnp.abs(z3)))
    f = (f1 + f2 + _ALPHA * f3) * (1.0 / (_BS * _L))
    acc = jnp.zeros((8, 128), jnp.float32)
    for rr in range(0, _ROWS, 8):
        acc = acc + f[rr:rr + 8]

    @pl.when(i == 0)
    def _init():
        o_ref[...] = jnp.zeros((8, 128), jnp.float32)

    o_ref[...] += acc


def _loss(counts, x3, t3, a1, a2):
    nprog = _BS * _CPM
    return pl.pallas_call(
        _loss_body,
        grid=(nprog,),
        in_specs=[
            pl.BlockSpec(memory_space=pltpu.SMEM),
            pl.BlockSpec((1, _ROWS, 128), lambda i: (i // _CPM, i % _CPM, 0)),
            pl.BlockSpec((1, _ROWS, 128), lambda i: (i // _CPM, i % _CPM, 0)),
            pl.BlockSpec(memory_space=pltpu.HBM),
            pl.BlockSpec(memory_space=pltpu.HBM),
        ],
        out_specs=pl.BlockSpec((8, 128), lambda i: (0, 0)),
        out_shape=jax.ShapeDtypeStruct((8, 128), jnp.float32),
        scratch_shapes=[
            pltpu.VMEM((_WINR, 128), jnp.float32),
            pltpu.VMEM((_WINR, 128), jnp.float32),
            pltpu.SemaphoreType.DMA,
            pltpu.SemaphoreType.DMA,
        ],
    )(counts, x3, t3, a1, a2)


def kernel(input, target):
    x3 = input.reshape(_BS, _CPM * _ROWS, 128)
    t3 = target.reshape(_BS, _CPM * _ROWS, 128)
    aug = _dilate(target.reshape(_BS, _H, _W))

    xf = input.reshape(_BS, _L)
    tf = target.reshape(_BS, _L)
    augf = aug.reshape(_BS, _L)
    a1, a2, counts = _compact_sc(xf, tf, augf)

    partials = _loss(counts, x3, t3, a1, a2)
    return jnp.sum(partials).reshape(())


# speedup vs baseline: 5.3001x; 1.2842x over previous
import functools

import jax
import jax.numpy as jnp
from jax import lax
from jax.experimental import pallas as pl
from jax.experimental.pallas import tpu as pltpu
from jax.experimental.pallas import tpu_sc as plsc

_BS, _H, _W = 8, 512, 512
_L = _H * _W
_ALPHA = 0.1
_POS_SEED = 5.0
_NEG_SEED = -5.0

_CHUNK = 8192
_CPM = _L // _CHUNK
_ROWS = _CHUNK // 128
_WINR = _ROWS + 2
_EXT = _CHUNK + 2 * 128
_WB = 2048
_WBMAX = (_L + _EXT + _WB - 1) // _WB
_ASTRIDE = _WBMAX * _WB
_AROWS = _ASTRIDE // 128


def _dilate_body(t_ref, o_ref):
    t2 = t_ref[0]
    h = t2
    for s in (1, 2, 3):
        zc = jnp.zeros((_H, s), jnp.float32)
        h = jnp.maximum(h, jnp.concatenate([t2[:, s:], zc], axis=1))
        h = jnp.maximum(h, jnp.concatenate([zc, t2[:, : _W - s]], axis=1))
    v = h
    for s in (1, 2, 3):
        zr = jnp.zeros((s, _W), jnp.float32)
        v = jnp.maximum(v, jnp.concatenate([h[s:, :], zr], axis=0))
        v = jnp.maximum(v, jnp.concatenate([zr, h[: _H - s, :]], axis=0))
    o_ref[0] = (v > 0.0).astype(jnp.float32)


def _dilate(t3):
    return pl.pallas_call(
        _dilate_body,
        grid=(_BS,),
        in_specs=[pl.BlockSpec((1, _H, _W), lambda m: (m, 0, 0))],
        out_specs=pl.BlockSpec((1, _H, _W), lambda m: (m, 0, 0)),
        out_shape=jax.ShapeDtypeStruct((_BS, _H, _W), jnp.float32),
    )(t3)


_NSC = 2
_SUBC = 16
_MAPS_PER_CORE = _BS // _NSC
_CHK = _L // _SUBC
_SUB = 8192
_NSUB = _CHK // _SUB
_VPS = _SUB // 16
_VROWS = _CHK // 128 + 2
_EXTR = (_EXT + 127) // 128
_EXTSLOTS = (_EXTR + _SUBC - 1) // _SUBC


def _sc_body(x_hbm, t_hbm, aug_hbm, a1_hbm, a2_hbm, cnt_hbm,
             xb, tb, gb, vpos, vfp, vneg, idxb, grow, srow, trow,
             crow, tbl, csp, a1sp, a2sp, sem):
    c = lax.axis_index("c")
    s = lax.axis_index("s")
    lane = lax.iota(jnp.int32, 16)
    dump = _L + _EXT

    def popcnt(mask):
        return plsc.cumsum(jnp.where(mask, 1, 0))[15]

    def scatter_local(valref, cnt, base, asp):
        nrows = (cnt + 127) // 128

        def mkrow(j, _):
            for v in range(8):
                p = j * 128 + v * 16 + lane
                iv = jnp.where(p < cnt, base + p, dump + (p & 63))
                idxb[j, pl.ds(v * 16, 16)] = iv
            return 0

        lax.fori_loop(0, nrows, mkrow, 0)

        def fire(j, _):
            pltpu.async_copy(valref.at[pl.ds(j * 128, 128)],
                             asp.at[idxb.at[j]], sem)
            return 0

        lax.fori_loop(0, nrows, fire, 0)

        def drain(j, _):
            pltpu.make_async_copy(valref.at[pl.ds(0, 128)],
                                  asp.at[idxb.at[0]], sem).wait()
            return 0

        lax.fori_loop(0, nrows, drain, 0)

    def extend(asp, n):
        def eloop(jj, _):
            row = s + jj * _SUBC

            @pl.when(row < _EXTR)
            def _():
                for v in range(8):
                    p = n + row * 128 + v * 16 + lane
                    grow[0, pl.ds(v * 16, 16)] = lax.rem(p, n)
                    srow[0, pl.ds(v * 16, 16)] = p
                pltpu.sync_copy(asp.at[grow.at[0]], trow.at[0])
                pltpu.sync_copy(trow.at[0], asp.at[srow.at[0]])
            return 0

        lax.fori_loop(0, _EXTSLOTS, eloop, 0)

    def seed(asp, n, value):
        @pl.when(jnp.logical_and(s == 0, n == 0))
        def _():
            for v in range(8):
                p = v * 16 + lane
                trow[0, pl.ds(v * 16, 16)] = jnp.full((16,), value,
                                                      jnp.float32)
                idxb[0, pl.ds(v * 16, 16)] = jnp.where(
                    p == 0, 0, dump + (p & 63))
            pltpu.sync_copy(trow.at[0], asp.at[idxb.at[0]])

    def writeback(asp, n, ahbm, abase):
        trips = (n + _EXT + _WB - 1) // _WB

        def wloop(jj, _):
            ch = s + jj * _SUBC

            @pl.when(ch < trips)
            def _():
                pltpu.sync_copy(asp.at[pl.ds(ch * _WB, _WB)],
                                ahbm.at[pl.ds(abase + ch * _WB, _WB)])
            return 0

        lax.fori_loop(0, (_WBMAX + _SUBC - 1) // _SUBC, wloop, 0)

    def per_map(mi, _unused):
        m = c * _MAPS_PER_CORE + mi
        gbase = m * _L + s * _CHK
        abase = m * _ASTRIDE

        def sub_loop(sub, carry):
            off = gbase + sub * _SUB
            pltpu.sync_copy(x_hbm.at[pl.ds(off, _SUB)], xb)
            pltpu.sync_copy(t_hbm.at[pl.ds(off, _SUB)], tb)
            pltpu.sync_copy(aug_hbm.at[pl.ds(off, _SUB)], gb)

            def vloop(v, carry2):
                w1, w2, w3 = carry2
                o = v * 16
                xv = xb[pl.ds(o, 16)]
                tv = tb[pl.ds(o, 16)]
                gv = gb[pl.ds(o, 16)]
                pos = tv > 0.0
                neg = gv == 0.0
                fp = jnp.logical_and(xv > 0.0, neg)
                packed = (jnp.where(pos, 1, 0) + jnp.where(fp, 1 << 10, 0)
                          + jnp.where(neg, 1 << 20, 0))
                pk = plsc.cumsum(packed)[15]
                plsc.store_compressed(vpos.at[pl.ds(w1, 16)], xv, mask=pos)
                plsc.store_compressed(vfp.at[pl.ds(w2, 16)], xv, mask=fp)
                plsc.store_compressed(vneg.at[pl.ds(w3, 16)], xv, mask=neg)
                return (w1 + (pk & 0x3FF), w2 + ((pk >> 10) & 0x3FF),
                        w3 + (pk >> 20))

            return lax.fori_loop(0, _VPS, vloop, carry, unroll=8)

        z0 = jnp.int32(0)
        wp1, wp2, wp3 = lax.fori_loop(0, _NSUB, sub_loop, (z0, z0, z0))

        crow[pl.ds(0, 16)] = jnp.where(
            lane == 0, wp1, jnp.where(lane == 1, wp2,
                                      jnp.where(lane == 2, wp3, 0)))
        pltpu.sync_copy(crow, csp.at[c, s])
        plsc.subcore_barrier()
        pltpu.sync_copy(csp.at[c], tbl)

        def offs(j, carry):
            o1, o2, o3, t1, t2, t3 = carry
            rv = tbl[j, pl.ds(0, 16)]
            v1 = rv[0]
            v2 = rv[1]
            v3 = rv[2]
            before = (j < s).astype(jnp.int32)
            return (o1 + before * v1, o2 + before * v2, o3 + before * v3,
                    t1 + v1, t2 + v2, t3 + v3)

        z = jnp.int32(0)
        o1, o2, o3, n1, nf, nn = lax.fori_loop(0, _SUBC, offs,
                                               (z, z, z, z, z, z))
        use_fp = nf > 0
        n2 = jnp.where(use_fp, nf, nn)
        o2c = jnp.where(use_fp, o2, o3)
        c2c = jnp.where(use_fp, wp2, wp3)

        scatter_local(vpos, wp1, o1, a1sp)

        @pl.when(use_fp)
        def _():
            scatter_local(vfp, c2c, o2c, a2sp)

        @pl.when(jnp.logical_not(use_fp))
        def _():
            scatter_local(vneg, c2c, o2c, a2sp)

        seed(a1sp, n1, _POS_SEED)
        seed(a2sp, n2, _NEG_SEED)
        n1e = jnp.maximum(n1, 1)
        n2e = jnp.maximum(n2, 1)
        plsc.subcore_barrier()

        extend(a1sp, n1e)
        extend(a2sp, n2e)
        plsc.subcore_barrier()

        writeback(a1sp, n1e, a1_hbm, abase)
        writeback(a2sp, n2e, a2_hbm, abase)

        @pl.when(s == 0)
        def _():
            crow[pl.ds(0, 16)] = jnp.where(
                lane == 0, n1e, jnp.where(lane == 1, n2e, 0))
            pltpu.sync_copy(crow, cnt_hbm.at[pl.ds(m * 16, 16)])

        plsc.subcore_barrier()
        return 0

    lax.fori_loop(0, _MAPS_PER_CORE, per_map, 0)


def _compact_sc(xf, tf, augf):
    mesh = plsc.VectorSubcoreMesh(core_axis_name="c", subcore_axis_name="s")
    a1, a2, cnt = pl.kernel(
        _sc_body,
        out_type=[
            jax.ShapeDtypeStruct((_BS * _ASTRIDE,), jnp.float32),
            jax.ShapeDtypeStruct((_BS * _ASTRIDE,), jnp.float32),
            jax.ShapeDtypeStruct((_BS * 16,), jnp.int32),
        ],
        mesh=mesh,
        compiler_params=pltpu.CompilerParams(needs_layout_passes=False),
        scratch_types=[
            pltpu.VMEM((_SUB,), jnp.float32),
            pltpu.VMEM((_SUB,), jnp.float32),
            pltpu.VMEM((_SUB,), jnp.float32),
            pltpu.VMEM((_VROWS * 128,), jnp.float32),
            pltpu.VMEM((_VROWS * 128,), jnp.float32),
            pltpu.VMEM((_VROWS * 128,), jnp.float32),
            pltpu.VMEM((_VROWS, 128), jnp.int32),
            pltpu.VMEM((1, 128), jnp.int32),
            pltpu.VMEM((1, 128), jnp.int32),
            pltpu.VMEM((1, 128), jnp.float32),
            pltpu.VMEM((16,), jnp.int32),
            pltpu.VMEM((_SUBC, 16), jnp.int32),
            pltpu.VMEM_SHARED((_NSC, _SUBC, 16), jnp.int32),
            pltpu.VMEM_SHARED((_ASTRIDE,), jnp.float32),
            pltpu.VMEM_SHARED((_ASTRIDE,), jnp.float32),
            pltpu.SemaphoreType.DMA,
        ],
    )(xf.reshape(-1), tf.reshape(-1), augf.reshape(-1))
    return (a1.reshape(_BS, _AROWS, 128), a2.reshape(_BS, _AROWS, 128),
            cnt.reshape(_BS, 16))


def _compact_jnp(xf, tf, augf):
    ar = jnp.arange(_L)

    def one(x, t, aug):
        pos = t > 0.0
        neg = aug == 0.0
        fp = jnp.logical_and(x > 0.0, neg)
        nfp = jnp.sum(fp.astype(jnp.int32))
        use_fp = nfp > 0
        cho = jnp.where(use_fp, fp, neg)

        def compact(mask, seed):
            n = jnp.sum(mask.astype(jnp.int32))
            key = jnp.where(mask, ar, ar + _L)
            vals = x[jnp.argsort(key)]
            vals = jnp.where(n == 0, vals.at[0].set(seed), vals)
            neff = jnp.maximum(n, 1)
            idx = jnp.arange(_ASTRIDE) % neff
            return vals[idx], neff

        a1, n1 = compact(pos, _POS_SEED)
        a2, n2 = compact(cho, _NEG_SEED)
        return a1, a2, n1, n2

    a1, a2, n1, n2 = jax.vmap(one)(xf, tf, augf)
    counts = jnp.zeros((_BS, 16), jnp.int32)
    counts = counts.at[:, 0].set(n1).at[:, 1].set(n2)
    return (a1.reshape(_BS, _AROWS, 128), a2.reshape(_BS, _AROWS, 128),
            counts)


def _loss_body(counts_ref, x_ref, t_ref, a1_hbm, a2_hbm, o_ref,
               w1, w2, sem1, sem2):
    i = pl.program_id(0)
    nprog = _BS * _CPM

    def win(idx, slot):
        m = idx // _CPM
        k = idx - m * _CPM
        s1 = lax.rem(k * _CHUNK, counts_ref[m, 0])
        s2 = lax.rem(k * _CHUNK, counts_ref[m, 1])
        c1 = pltpu.make_async_copy(a1_hbm.at[m, pl.ds(s1 // 128, _WINR)],
                                   w1.at[slot], sem1.at[slot])
        c2 = pltpu.make_async_copy(a2_hbm.at[m, pl.ds(s2 // 128, _WINR)],
                                   w2.at[slot], sem2.at[slot])
        return c1, c2, lax.rem(s1, 128), lax.rem(s2, 128)

    slot = lax.rem(i, 2)

    @pl.when(i == 0)
    def _():
        c1, c2, _r1, _r2 = win(i, slot)
        c1.start()
        c2.start()

    @pl.when(i + 1 < nprog)
    def _():
        c1, c2, _r1, _r2 = win(i + 1, lax.rem(i + 1, 2))
        c1.start()
        c2.start()

    c1, c2, r1, r2 = win(i, slot)
    c1.wait()
    c2.wait()

    m = i // _CPM
    lane = lax.broadcasted_iota(jnp.int32, (_ROWS, 128), 1)

    def unshift(w_ref, r):
        wv = w_ref[slot]
        u = pltpu.roll(wv, lax.rem(128 - r, 128), axis=1)
        return jnp.where(lane < 128 - r, u[0:_ROWS], u[1:_ROWS + 1])

    g1 = unshift(w1, r1)
    g2 = unshift(w2, r2)

    x = x_ref[0]
    t = t_ref[0]
    z1 = g1 * x
    f1 = jnp.maximum(z1, 0.0) - z1 * t + jnp.log1p(jnp.exp(-jnp.abs(z1)))
    f2 = jnp.maximum(-g1, 0.0) + jnp.log1p(jnp.exp(-jnp.abs(g1)))
    z3 = g1 * g2
    f3 = jnp.maximum(z3, 0.0) + jnp.log1p(jnp.exp(-jnp.abs(z3)))
    f = (f1 + f2 + _ALPHA * f3) * (1.0 / (_BS * _L))
    acc = jnp.zeros((8, 128), jnp.float32)
    for rr in range(0, _ROWS, 8):
        acc = acc + f[rr:rr + 8]

    @pl.when(i == 0)
    def _init():
        o_ref[...] = jnp.zeros((8, 128), jnp.float32)

    o_ref[...] += acc


def _loss(counts, x3, t3, a1, a2):
    nprog = _BS * _CPM
    return pl.pallas_call(
        _loss_body,
        grid=(nprog,),
        in_specs=[
            pl.BlockSpec(memory_space=pltpu.SMEM),
            pl.BlockSpec((1, _ROWS, 128), lambda i: (i // _CPM, i % _CPM, 0)),
            pl.BlockSpec((1, _ROWS, 128), lambda i: (i // _CPM, i % _CPM, 0)),
            pl.BlockSpec(memory_space=pltpu.HBM),
            pl.BlockSpec(memory_space=pltpu.HBM),
        ],
        out_specs=pl.BlockSpec((8, 128), lambda i: (0, 0)),
        out_shape=jax.ShapeDtypeStruct((8, 128), jnp.float32),
        scratch_shapes=[
            pltpu.VMEM((2, _WINR, 128), jnp.float32),
            pltpu.VMEM((2, _WINR, 128), jnp.float32),
            pltpu.SemaphoreType.DMA((2,)),
            pltpu.SemaphoreType.DMA((2,)),
        ],
    )(counts, x3, t3, a1, a2)


def kernel(input, target):
    x3 = input.reshape(_BS, _CPM * _ROWS, 128)
    t3 = target.reshape(_BS, _CPM * _ROWS, 128)
    aug = _dilate(target.reshape(_BS, _H, _W))

    xf = input.reshape(_BS, _L)
    tf = target.reshape(_BS, _L)
    augf = aug.reshape(_BS, _L)
    a1, a2, counts = _compact_sc(xf, tf, augf)

    partials = _loss(counts, x3, t3, a1, a2)
    return jnp.sum(partials).reshape(())


# speedup vs baseline: 5.3152x; 1.0028x over previous
import functools

import jax
import jax.numpy as jnp
from jax import lax
from jax.experimental import pallas as pl
from jax.experimental.pallas import tpu as pltpu
from jax.experimental.pallas import tpu_sc as plsc

_BS, _H, _W = 8, 512, 512
_L = _H * _W
_ALPHA = 0.1
_POS_SEED = 5.0
_NEG_SEED = -5.0

_CHUNK = 8192
_CPM = _L // _CHUNK
_ROWS = _CHUNK // 128
_WINR = _ROWS + 2
_EXT = _CHUNK + 2 * 128
_WB = 2048
_WBMAX = (_L + _EXT + _WB - 1) // _WB
_ASTRIDE = _WBMAX * _WB
_AROWS = _ASTRIDE // 128


def _dilate_body(t_ref, o_ref):
    t2 = t_ref[0]
    h = t2
    for s in (1, 2, 3):
        zc = jnp.zeros((_H, s), jnp.float32)
        h = jnp.maximum(h, jnp.concatenate([t2[:, s:], zc], axis=1))
        h = jnp.maximum(h, jnp.concatenate([zc, t2[:, : _W - s]], axis=1))
    v = h
    for s in (1, 2, 3):
        zr = jnp.zeros((s, _W), jnp.float32)
        v = jnp.maximum(v, jnp.concatenate([h[s:, :], zr], axis=0))
        v = jnp.maximum(v, jnp.concatenate([zr, h[: _H - s, :]], axis=0))
    o_ref[0] = (v > 0.0).astype(jnp.float32)


def _dilate(t3):
    return pl.pallas_call(
        _dilate_body,
        grid=(_BS,),
        in_specs=[pl.BlockSpec((1, _H, _W), lambda m: (m, 0, 0))],
        out_specs=pl.BlockSpec((1, _H, _W), lambda m: (m, 0, 0)),
        out_shape=jax.ShapeDtypeStruct((_BS, _H, _W), jnp.float32),
    )(t3)


_NSC = 2
_SUBC = 16
_MAPS_PER_CORE = _BS // _NSC
_CHK = _L // _SUBC
_SUB = 8192
_NSUB = _CHK // _SUB
_VPS = _SUB // 16
_VROWS = _CHK // 128 + 2
_EXTR = (_EXT + 127) // 128
_EXTSLOTS = (_EXTR + _SUBC - 1) // _SUBC


def _sc_body(x_hbm, t_hbm, aug_hbm, a1_hbm, a2_hbm, cnt_hbm,
             xb, tb, gb, vpos, vfp, vneg, idxb, grow, srow, trow,
             crow, tbl, csp, a1sp, a2sp, sem):
    c = lax.axis_index("c")
    s = lax.axis_index("s")
    lane = lax.iota(jnp.int32, 16)
    dump = _L + _EXT

    def popcnt(mask):
        return plsc.cumsum(jnp.where(mask, 1, 0))[15]

    def scatter_local(valref, cnt, base, asp):
        nrows = (cnt + 127) // 128

        def mkrow(j, _):
            for v in range(8):
                p = j * 128 + v * 16 + lane
                iv = jnp.where(p < cnt, base + p, dump + (p & 63))
                idxb[j, pl.ds(v * 16, 16)] = iv
            return 0

        lax.fori_loop(0, nrows, mkrow, 0)

        def fire(j, _):
            pltpu.async_copy(valref.at[pl.ds(j * 128, 128)],
                             asp.at[idxb.at[j]], sem)
            return 0

        lax.fori_loop(0, nrows, fire, 0)

        def drain(j, _):
            pltpu.make_async_copy(valref.at[pl.ds(0, 128)],
                                  asp.at[idxb.at[0]], sem).wait()
            return 0

        lax.fori_loop(0, nrows, drain, 0)

    def extend(asp, n):
        def eloop(jj, _):
            row = s + jj * _SUBC

            @pl.when(row < _EXTR)
            def _():
                for v in range(8):
                    p = n + row * 128 + v * 16 + lane
                    grow[0, pl.ds(v * 16, 16)] = lax.rem(p, n)
                    srow[0, pl.ds(v * 16, 16)] = p
                pltpu.sync_copy(asp.at[grow.at[0]], trow.at[0])
                pltpu.sync_copy(trow.at[0], asp.at[srow.at[0]])
            return 0

        lax.fori_loop(0, _EXTSLOTS, eloop, 0)

    def seed(asp, n, value):
        @pl.when(jnp.logical_and(s == 0, n == 0))
        def _():
            for v in range(8):
                p = v * 16 + lane
                trow[0, pl.ds(v * 16, 16)] = jnp.full((16,), value,
                                                      jnp.float32)
                idxb[0, pl.ds(v * 16, 16)] = jnp.where(
                    p == 0, 0, dump + (p & 63))
            pltpu.sync_copy(trow.at[0], asp.at[idxb.at[0]])

    def writeback(asp, n, ahbm, abase):
        trips = (n + _EXT + _WB - 1) // _WB

        def wloop(jj, _):
            ch = s + jj * _SUBC

            @pl.when(ch < trips)
            def _():
                pltpu.sync_copy(asp.at[pl.ds(ch * _WB, _WB)],
                                ahbm.at[pl.ds(abase + ch * _WB, _WB)])
            return 0

        lax.fori_loop(0, (_WBMAX + _SUBC - 1) // _SUBC, wloop, 0)

    def per_map(mi, _unused):
        m = c * _MAPS_PER_CORE + mi
        gbase = m * _L + s * _CHK
        abase = m * _ASTRIDE

        def sub_loop(sub, carry):
            off = gbase + sub * _SUB
            pltpu.sync_copy(x_hbm.at[pl.ds(off, _SUB)], xb)
            pltpu.sync_copy(t_hbm.at[pl.ds(off, _SUB)], tb)
            pltpu.sync_copy(aug_hbm.at[pl.ds(off, _SUB)], gb)

            def vloop(v, carry2):
                w1, w2, w3 = carry2
                o = v * 16
                xv = xb[pl.ds(o, 16)]
                tv = tb[pl.ds(o, 16)]
                gv = gb[pl.ds(o, 16)]
                pos = tv > 0.0
                neg = gv == 0.0
                fp = jnp.logical_and(xv > 0.0, neg)
                packed = (jnp.where(pos, 1, 0) + jnp.where(fp, 1 << 10, 0)
                          + jnp.where(neg, 1 << 20, 0))
                pk = plsc.cumsum(packed)[15]
                plsc.store_compressed(vpos.at[pl.ds(w1, 16)], xv, mask=pos)
                plsc.store_compressed(vfp.at[pl.ds(w2, 16)], xv, mask=fp)
                plsc.store_compressed(vneg.at[pl.ds(w3, 16)], xv, mask=neg)
                return (w1 + (pk & 0x3FF), w2 + ((pk >> 10) & 0x3FF),
                        w3 + (pk >> 20))

            return lax.fori_loop(0, _VPS, vloop, carry, unroll=8)

        z0 = jnp.int32(0)
        wp1, wp2, wp3 = lax.fori_loop(0, _NSUB, sub_loop, (z0, z0, z0))

        crow[pl.ds(0, 16)] = jnp.where(
            lane == 0, wp1, jnp.where(lane == 1, wp2,
                                      jnp.where(lane == 2, wp3, 0)))
        pltpu.sync_copy(crow, csp.at[c, s])
        plsc.subcore_barrier()
        pltpu.sync_copy(csp.at[c], tbl)

        def offs(j, carry):
            o1, o2, o3, t1, t2, t3 = carry
            rv = tbl[j, pl.ds(0, 16)]
            v1 = rv[0]
            v2 = rv[1]
            v3 = rv[2]
            before = (j < s).astype(jnp.int32)
            return (o1 + before * v1, o2 + before * v2, o3 + before * v3,
                    t1 + v1, t2 + v2, t3 + v3)

        z = jnp.int32(0)
        o1, o2, o3, n1, nf, nn = lax.fori_loop(0, _SUBC, offs,
                                               (z, z, z, z, z, z))
        use_fp = nf > 0
        n2 = jnp.where(use_fp, nf, nn)
        o2c = jnp.where(use_fp, o2, o3)
        c2c = jnp.where(use_fp, wp2, wp3)

        scatter_local(vpos, wp1, o1, a1sp)

        @pl.when(use_fp)
        def _():
            scatter_local(vfp, c2c, o2c, a2sp)

        @pl.when(jnp.logical_not(use_fp))
        def _():
            scatter_local(vneg, c2c, o2c, a2sp)

        seed(a1sp, n1, _POS_SEED)
        seed(a2sp, n2, _NEG_SEED)
        n1e = jnp.maximum(n1, 1)
        n2e = jnp.maximum(n2, 1)
        plsc.subcore_barrier()

        extend(a1sp, n1e)
        extend(a2sp, n2e)
        plsc.subcore_barrier()

        writeback(a1sp, n1e, a1_hbm, abase)
        writeback(a2sp, n2e, a2_hbm, abase)

        @pl.when(s == 0)
        def _():
            crow[pl.ds(0, 16)] = jnp.where(
                lane == 0, n1e, jnp.where(lane == 1, n2e, 0))
            pltpu.sync_copy(crow, cnt_hbm.at[pl.ds(m * 16, 16)])

        plsc.subcore_barrier()
        return 0

    lax.fori_loop(0, _MAPS_PER_CORE, per_map, 0)


def _compact_sc(xf, tf, augf):
    mesh = plsc.VectorSubcoreMesh(core_axis_name="c", subcore_axis_name="s")
    a1, a2, cnt = pl.kernel(
        _sc_body,
        out_type=[
            jax.ShapeDtypeStruct((_BS * _ASTRIDE,), jnp.float32),
            jax.ShapeDtypeStruct((_BS * _ASTRIDE,), jnp.float32),
            jax.ShapeDtypeStruct((_BS * 16,), jnp.int32),
        ],
        mesh=mesh,
        compiler_params=pltpu.CompilerParams(needs_layout_passes=False),
        scratch_types=[
            pltpu.VMEM((_SUB,), jnp.float32),
            pltpu.VMEM((_SUB,), jnp.float32),
            pltpu.VMEM((_SUB,), jnp.float32),
            pltpu.VMEM((_VROWS * 128,), jnp.float32),
            pltpu.VMEM((_VROWS * 128,), jnp.float32),
            pltpu.VMEM((_VROWS * 128,), jnp.float32),
            pltpu.VMEM((_VROWS, 128), jnp.int32),
            pltpu.VMEM((1, 128), jnp.int32),
            pltpu.VMEM((1, 128), jnp.int32),
            pltpu.VMEM((1, 128), jnp.float32),
            pltpu.VMEM((16,), jnp.int32),
            pltpu.VMEM((_SUBC, 16), jnp.int32),
            pltpu.VMEM_SHARED((_NSC, _SUBC, 16), jnp.int32),
            pltpu.VMEM_SHARED((_ASTRIDE,), jnp.float32),
            pltpu.VMEM_SHARED((_ASTRIDE,), jnp.float32),
            pltpu.SemaphoreType.DMA,
        ],
    )(xf.reshape(-1), tf.reshape(-1), augf.reshape(-1))
    return (a1.reshape(_BS, _AROWS, 128), a2.reshape(_BS, _AROWS, 128),
            cnt.reshape(_BS, 16))


def _compact_jnp(xf, tf, augf):
    ar = jnp.arange(_L)

    def one(x, t, aug):
        pos = t > 0.0
        neg = aug == 0.0
        fp = jnp.logical_and(x > 0.0, neg)
        nfp = jnp.sum(fp.astype(jnp.int32))
        use_fp = nfp > 0
        cho = jnp.where(use_fp, fp, neg)

        def compact(mask, seed):
            n = jnp.sum(mask.astype(jnp.int32))
            key = jnp.where(mask, ar, ar + _L)
            vals = x[jnp.argsort(key)]
            vals = jnp.where(n == 0, vals.at[0].set(seed), vals)
            neff = jnp.maximum(n, 1)
            idx = jnp.arange(_ASTRIDE) % neff
            return vals[idx], neff

        a1, n1 = compact(pos, _POS_SEED)
        a2, n2 = compact(cho, _NEG_SEED)
        return a1, a2, n1, n2

    a1, a2, n1, n2 = jax.vmap(one)(xf, tf, augf)
    counts = jnp.zeros((_BS, 16), jnp.int32)
    counts = counts.at[:, 0].set(n1).at[:, 1].set(n2)
    return (a1.reshape(_BS, _AROWS, 128), a2.reshape(_BS, _AROWS, 128),
            counts)


def _loss_body(counts_ref, x_ref, t_ref, a1_hbm, a2_hbm, o_ref,
               w1, w2, sem1, sem2):
    i = pl.program_id(0)
    nprog = _BS * _CPM

    def win(idx, slot):
        m = idx // _CPM
        k = idx - m * _CPM
        s1 = lax.rem(k * _CHUNK, counts_ref[m, 0])
        s2 = lax.rem(k * _CHUNK, counts_ref[m, 1])
        c1 = pltpu.make_async_copy(a1_hbm.at[m, pl.ds(s1 // 128, _WINR)],
                                   w1.at[slot], sem1.at[slot])
        c2 = pltpu.make_async_copy(a2_hbm.at[m, pl.ds(s2 // 128, _WINR)],
                                   w2.at[slot], sem2.at[slot])
        return c1, c2, lax.rem(s1, 128), lax.rem(s2, 128)

    slot = lax.rem(i, 2)

    @pl.when(i == 0)
    def _():
        c1, c2, _r1, _r2 = win(i, slot)
        c1.start()
        c2.start()

    @pl.when(i + 1 < nprog)
    def _():
        c1, c2, _r1, _r2 = win(i + 1, lax.rem(i + 1, 2))
        c1.start()
        c2.start()

    c1, c2, r1, r2 = win(i, slot)
    c1.wait()
    c2.wait()

    m = i // _CPM
    lane = lax.broadcasted_iota(jnp.int32, (_ROWS, 128), 1)

    def unshift(w_ref, r):
        wv = w_ref[slot]
        u = pltpu.roll(wv, lax.rem(128 - r, 128), axis=1)
        return jnp.where(lane < 128 - r, u[0:_ROWS], u[1:_ROWS + 1])

    g1 = unshift(w1, r1)
    g2 = unshift(w2, r2)

    x = x_ref[0]
    t = t_ref[0]
    z1 = g1 * x
    z3 = g1 * g2
    e1 = jnp.exp(-jnp.abs(z1))
    e2 = jnp.exp(-jnp.abs(g1))
    flin = (jnp.maximum(z1, 0.0) - z1 * t + jnp.maximum(-g1, 0.0)
            + _ALPHA * jnp.maximum(z3, 0.0))
    flog = jnp.log((1.0 + e1) * (1.0 + e2)) \
        + _ALPHA * jnp.log1p(jnp.exp(-jnp.abs(z3)))
    f = (flin + flog) * (1.0 / (_BS * _L))
    acc = jnp.zeros((8, 128), jnp.float32)
    for rr in range(0, _ROWS, 8):
        acc = acc + f[rr:rr + 8]

    @pl.when(i == 0)
    def _init():
        o_ref[...] = jnp.zeros((8, 128), jnp.float32)

    o_ref[...] += acc


def _loss(counts, x3, t3, a1, a2):
    nprog = _BS * _CPM
    return pl.pallas_call(
        _loss_body,
        grid=(nprog,),
        in_specs=[
            pl.BlockSpec(memory_space=pltpu.SMEM),
            pl.BlockSpec((1, _ROWS, 128), lambda i: (i // _CPM, i % _CPM, 0)),
            pl.BlockSpec((1, _ROWS, 128), lambda i: (i // _CPM, i % _CPM, 0)),
            pl.BlockSpec(memory_space=pltpu.HBM),
            pl.BlockSpec(memory_space=pltpu.HBM),
        ],
        out_specs=pl.BlockSpec((8, 128), lambda i: (0, 0)),
        out_shape=jax.ShapeDtypeStruct((8, 128), jnp.float32),
        scratch_shapes=[
            pltpu.VMEM((2, _WINR, 128), jnp.float32),
            pltpu.VMEM((2, _WINR, 128), jnp.float32),
            pltpu.SemaphoreType.DMA((2,)),
            pltpu.SemaphoreType.DMA((2,)),
        ],
    )(counts, x3, t3, a1, a2)


def kernel(input, target):
    x3 = input.reshape(_BS, _CPM * _ROWS, 128)
    t3 = target.reshape(_BS, _CPM * _ROWS, 128)
    aug = _dilate(target.reshape(_BS, _H, _W))

    xf = input.reshape(_BS, _L)
    tf = target.reshape(_BS, _L)
    augf = aug.reshape(_BS, _L)
    a1, a2, counts = _compact_sc(xf, tf, augf)

    partials = _loss(counts, x3, t3, a1, a2)
    return jnp.sum(partials).reshape(())


# speedup vs baseline: 6.2315x; 1.1724x over previous
import functools

import jax
import jax.numpy as jnp
from jax import lax
from jax.experimental import pallas as pl
from jax.experimental.pallas import tpu as pltpu
from jax.experimental.pallas import tpu_sc as plsc

_BS, _H, _W = 8, 512, 512
_L = _H * _W
_ALPHA = 0.1
_POS_SEED = 5.0
_NEG_SEED = -5.0

_CHUNK = 16384
_CPM = _L // _CHUNK
_ROWS = _CHUNK // 128
_WINR = _ROWS + 2
_EXT = _CHUNK + 2 * 128
_WB = 2048
_WBMAX = (_L + _EXT + _WB - 1) // _WB
_ASTRIDE = _WBMAX * _WB
_AROWS = _ASTRIDE // 128


def _dilate_body(t_ref, o_ref):
    t2 = t_ref[0]
    h = t2
    for s in (1, 2, 3):
        zc = jnp.zeros((_H, s), jnp.float32)
        h = jnp.maximum(h, jnp.concatenate([t2[:, s:], zc], axis=1))
        h = jnp.maximum(h, jnp.concatenate([zc, t2[:, : _W - s]], axis=1))
    v = h
    for s in (1, 2, 3):
        zr = jnp.zeros((s, _W), jnp.float32)
        v = jnp.maximum(v, jnp.concatenate([h[s:, :], zr], axis=0))
        v = jnp.maximum(v, jnp.concatenate([zr, h[: _H - s, :]], axis=0))
    o_ref[0] = (v > 0.0).astype(jnp.float32)


def _dilate(t3):
    return pl.pallas_call(
        _dilate_body,
        grid=(_BS,),
        in_specs=[pl.BlockSpec((1, _H, _W), lambda m: (m, 0, 0))],
        out_specs=pl.BlockSpec((1, _H, _W), lambda m: (m, 0, 0)),
        out_shape=jax.ShapeDtypeStruct((_BS, _H, _W), jnp.float32),
    )(t3)


_NSC = 2
_SUBC = 16
_MAPS_PER_CORE = _BS // _NSC
_CHK = _L // _SUBC
_SUB = 8192
_NSUB = _CHK // _SUB
_VPS = _SUB // 16
_VROWS = _CHK // 128 + 2
_EXTR = (_EXT + 127) // 128
_EXTSLOTS = (_EXTR + _SUBC - 1) // _SUBC


def _sc_body(x_hbm, t_hbm, aug_hbm, a1_hbm, a2_hbm, cnt_hbm,
             xb, tb, gb, vpos, vfp, vneg, idxb, grow, srow, trow,
             crow, tbl, csp, a1sp, a2sp, sem):
    c = lax.axis_index("c")
    s = lax.axis_index("s")
    lane = lax.iota(jnp.int32, 16)
    dump = _L + _EXT

    def popcnt(mask):
        return plsc.cumsum(jnp.where(mask, 1, 0))[15]

    def scatter_local(valref, cnt, base, asp):
        nrows = (cnt + 127) // 128

        def mkrow(j, _):
            for v in range(8):
                p = j * 128 + v * 16 + lane
                iv = jnp.where(p < cnt, base + p, dump + (p & 63))
                idxb[j, pl.ds(v * 16, 16)] = iv
            return 0

        lax.fori_loop(0, nrows, mkrow, 0)

        def fire(j, _):
            pltpu.async_copy(valref.at[pl.ds(j * 128, 128)],
                             asp.at[idxb.at[j]], sem)
            return 0

        lax.fori_loop(0, nrows, fire, 0)

        def drain(j, _):
            pltpu.make_async_copy(valref.at[pl.ds(0, 128)],
                                  asp.at[idxb.at[0]], sem).wait()
            return 0

        lax.fori_loop(0, nrows, drain, 0)

    def extend(asp, n):
        def eloop(jj, _):
            row = s + jj * _SUBC

            @pl.when(row < _EXTR)
            def _():
                for v in range(8):
                    p = n + row * 128 + v * 16 + lane
                    grow[0, pl.ds(v * 16, 16)] = lax.rem(p, n)
                    srow[0, pl.ds(v * 16, 16)] = p
                pltpu.sync_copy(asp.at[grow.at[0]], trow.at[0])
                pltpu.sync_copy(trow.at[0], asp.at[srow.at[0]])
            return 0

        lax.fori_loop(0, _EXTSLOTS, eloop, 0)

    def seed(asp, n, value):
        @pl.when(jnp.logical_and(s == 0, n == 0))
        def _():
            for v in range(8):
                p = v * 16 + lane
                trow[0, pl.ds(v * 16, 16)] = jnp.full((16,), value,
                                                      jnp.float32)
                idxb[0, pl.ds(v * 16, 16)] = jnp.where(
                    p == 0, 0, dump + (p & 63))
            pltpu.sync_copy(trow.at[0], asp.at[idxb.at[0]])

    def writeback(asp, n, ahbm, abase):
        trips = (n + _EXT + _WB - 1) // _WB

        def wloop(jj, _):
            ch = s + jj * _SUBC

            @pl.when(ch < trips)
            def _():
                pltpu.sync_copy(asp.at[pl.ds(ch * _WB, _WB)],
                                ahbm.at[pl.ds(abase + ch * _WB, _WB)])
            return 0

        lax.fori_loop(0, (_WBMAX + _SUBC - 1) // _SUBC, wloop, 0)

    def per_map(mi, _unused):
        m = c * _MAPS_PER_CORE + mi
        gbase = m * _L + s * _CHK
        abase = m * _ASTRIDE

        def sub_loop(sub, carry):
            off = gbase + sub * _SUB
            pltpu.sync_copy(x_hbm.at[pl.ds(off, _SUB)], xb)
            pltpu.sync_copy(t_hbm.at[pl.ds(off, _SUB)], tb)
            pltpu.sync_copy(aug_hbm.at[pl.ds(off, _SUB)], gb)

            def vloop(v, carry2):
                w1, w2, w3 = carry2
                o = v * 16
                xv = xb[pl.ds(o, 16)]
                tv = tb[pl.ds(o, 16)]
                gv = gb[pl.ds(o, 16)]
                pos = tv > 0.0
                neg = gv == 0.0
                fp = jnp.logical_and(xv > 0.0, neg)
                packed = (jnp.where(pos, 1, 0) + jnp.where(fp, 1 << 10, 0)
                          + jnp.where(neg, 1 << 20, 0))
                pk = plsc.cumsum(packed)[15]
                plsc.store_compressed(vpos.at[pl.ds(w1, 16)], xv, mask=pos)
                plsc.store_compressed(vfp.at[pl.ds(w2, 16)], xv, mask=fp)
                plsc.store_compressed(vneg.at[pl.ds(w3, 16)], xv, mask=neg)
                return (w1 + (pk & 0x3FF), w2 + ((pk >> 10) & 0x3FF),
                        w3 + (pk >> 20))

            return lax.fori_loop(0, _VPS, vloop, carry, unroll=8)

        z0 = jnp.int32(0)
        wp1, wp2, wp3 = lax.fori_loop(0, _NSUB, sub_loop, (z0, z0, z0))

        crow[pl.ds(0, 16)] = jnp.where(
            lane == 0, wp1, jnp.where(lane == 1, wp2,
                                      jnp.where(lane == 2, wp3, 0)))
        pltpu.sync_copy(crow, csp.at[c, s])
        plsc.subcore_barrier()
        pltpu.sync_copy(csp.at[c], tbl)

        def offs(j, carry):
            o1, o2, o3, t1, t2, t3 = carry
            rv = tbl[j, pl.ds(0, 16)]
            v1 = rv[0]
            v2 = rv[1]
            v3 = rv[2]
            before = (j < s).astype(jnp.int32)
            return (o1 + before * v1, o2 + before * v2, o3 + before * v3,
                    t1 + v1, t2 + v2, t3 + v3)

        z = jnp.int32(0)
        o1, o2, o3, n1, nf, nn = lax.fori_loop(0, _SUBC, offs,
                                               (z, z, z, z, z, z))
        use_fp = nf > 0
        n2 = jnp.where(use_fp, nf, nn)
        o2c = jnp.where(use_fp, o2, o3)
        c2c = jnp.where(use_fp, wp2, wp3)

        scatter_local(vpos, wp1, o1, a1sp)

        @pl.when(use_fp)
        def _():
            scatter_local(vfp, c2c, o2c, a2sp)

        @pl.when(jnp.logical_not(use_fp))
        def _():
            scatter_local(vneg, c2c, o2c, a2sp)

        seed(a1sp, n1, _POS_SEED)
        seed(a2sp, n2, _NEG_SEED)
        n1e = jnp.maximum(n1, 1)
        n2e = jnp.maximum(n2, 1)
        plsc.subcore_barrier()

        extend(a1sp, n1e)
        extend(a2sp, n2e)
        plsc.subcore_barrier()

        writeback(a1sp, n1e, a1_hbm, abase)
        writeback(a2sp, n2e, a2_hbm, abase)

        @pl.when(s == 0)
        def _():
            crow[pl.ds(0, 16)] = jnp.where(
                lane == 0, n1e, jnp.where(lane == 1, n2e, 0))
            pltpu.sync_copy(crow, cnt_hbm.at[pl.ds(m * 16, 16)])

        plsc.subcore_barrier()
        return 0

    lax.fori_loop(0, _MAPS_PER_CORE, per_map, 0)


def _compact_sc(xf, tf, augf):
    mesh = plsc.VectorSubcoreMesh(core_axis_name="c", subcore_axis_name="s")
    a1, a2, cnt = pl.kernel(
        _sc_body,
        out_type=[
            jax.ShapeDtypeStruct((_BS * _ASTRIDE,), jnp.float32),
            jax.ShapeDtypeStruct((_BS * _ASTRIDE,), jnp.float32),
            jax.ShapeDtypeStruct((_BS * 16,), jnp.int32),
        ],
        mesh=mesh,
        compiler_params=pltpu.CompilerParams(needs_layout_passes=False),
        scratch_types=[
            pltpu.VMEM((_SUB,), jnp.float32),
            pltpu.VMEM((_SUB,), jnp.float32),
            pltpu.VMEM((_SUB,), jnp.float32),
            pltpu.VMEM((_VROWS * 128,), jnp.float32),
            pltpu.VMEM((_VROWS * 128,), jnp.float32),
            pltpu.VMEM((_VROWS * 128,), jnp.float32),
            pltpu.VMEM((_VROWS, 128), jnp.int32),
            pltpu.VMEM((1, 128), jnp.int32),
            pltpu.VMEM((1, 128), jnp.int32),
            pltpu.VMEM((1, 128), jnp.float32),
            pltpu.VMEM((16,), jnp.int32),
            pltpu.VMEM((_SUBC, 16), jnp.int32),
            pltpu.VMEM_SHARED((_NSC, _SUBC, 16), jnp.int32),
            pltpu.VMEM_SHARED((_ASTRIDE,), jnp.float32),
            pltpu.VMEM_SHARED((_ASTRIDE,), jnp.float32),
            pltpu.SemaphoreType.DMA,
        ],
    )(xf.reshape(-1), tf.reshape(-1), augf.reshape(-1))
    return (a1.reshape(_BS, _AROWS, 128), a2.reshape(_BS, _AROWS, 128),
            cnt.reshape(_BS, 16))


def _compact_jnp(xf, tf, augf):
    ar = jnp.arange(_L)

    def one(x, t, aug):
        pos = t > 0.0
        neg = aug == 0.0
        fp = jnp.logical_and(x > 0.0, neg)
        nfp = jnp.sum(fp.astype(jnp.int32))
        use_fp = nfp > 0
        cho = jnp.where(use_fp, fp, neg)

        def compact(mask, seed):
            n = jnp.sum(mask.astype(jnp.int32))
            key = jnp.where(mask, ar, ar + _L)
            vals = x[jnp.argsort(key)]
            vals = jnp.where(n == 0, vals.at[0].set(seed), vals)
            neff = jnp.maximum(n, 1)
            idx = jnp.arange(_ASTRIDE) % neff
            return vals[idx], neff

        a1, n1 = compact(pos, _POS_SEED)
        a2, n2 = compact(cho, _NEG_SEED)
        return a1, a2, n1, n2

    a1, a2, n1, n2 = jax.vmap(one)(xf, tf, augf)
    counts = jnp.zeros((_BS, 16), jnp.int32)
    counts = counts.at[:, 0].set(n1).at[:, 1].set(n2)
    return (a1.reshape(_BS, _AROWS, 128), a2.reshape(_BS, _AROWS, 128),
            counts)


def _loss_body(counts_ref, x_ref, t_ref, a1_hbm, a2_hbm, o_ref,
               w1, w2, sem1, sem2):
    i = pl.program_id(0)
    nprog = _BS * _CPM

    def win(idx, slot):
        m = idx // _CPM
        k = idx - m * _CPM
        s1 = lax.rem(k * _CHUNK, counts_ref[m, 0])
        s2 = lax.rem(k * _CHUNK, counts_ref[m, 1])
        c1 = pltpu.make_async_copy(a1_hbm.at[m, pl.ds(s1 // 128, _WINR)],
                                   w1.at[slot], sem1.at[slot])
        c2 = pltpu.make_async_copy(a2_hbm.at[m, pl.ds(s2 // 128, _WINR)],
                                   w2.at[slot], sem2.at[slot])
        return c1, c2, lax.rem(s1, 128), lax.rem(s2, 128)

    slot = lax.rem(i, 2)

    @pl.when(i == 0)
    def _():
        c1, c2, _r1, _r2 = win(i, slot)
        c1.start()
        c2.start()

    @pl.when(i + 1 < nprog)
    def _():
        c1, c2, _r1, _r2 = win(i + 1, lax.rem(i + 1, 2))
        c1.start()
        c2.start()

    c1, c2, r1, r2 = win(i, slot)
    c1.wait()
    c2.wait()

    m = i // _CPM
    lane = lax.broadcasted_iota(jnp.int32, (_ROWS, 128), 1)

    def unshift(w_ref, r):
        wv = w_ref[slot]
        u = pltpu.roll(wv, lax.rem(128 - r, 128), axis=1)
        return jnp.where(lane < 128 - r, u[0:_ROWS], u[1:_ROWS + 1])

    g1 = unshift(w1, r1)
    g2 = unshift(w2, r2)

    x = x_ref[0]
    t = t_ref[0]
    z1 = g1 * x
    z3 = g1 * g2
    e1 = jnp.exp(-jnp.abs(z1))
    e2 = jnp.exp(-jnp.abs(g1))
    flin = (jnp.maximum(z1, 0.0) - z1 * t + jnp.maximum(-g1, 0.0)
            + _ALPHA * jnp.maximum(z3, 0.0))
    flog = jnp.log((1.0 + e1) * (1.0 + e2)) \
        + _ALPHA * jnp.log1p(jnp.exp(-jnp.abs(z3)))
    f = (flin + flog) * (1.0 / (_BS * _L))
    acc = jnp.zeros((8, 128), jnp.float32)
    for rr in range(0, _ROWS, 8):
        acc = acc + f[rr:rr + 8]

    @pl.when(i == 0)
    def _init():
        o_ref[...] = jnp.zeros((8, 128), jnp.float32)

    o_ref[...] += acc


def _loss(counts, x3, t3, a1, a2):
    nprog = _BS * _CPM
    return pl.pallas_call(
        _loss_body,
        grid=(nprog,),
        in_specs=[
            pl.BlockSpec(memory_space=pltpu.SMEM),
            pl.BlockSpec((1, _ROWS, 128), lambda i: (i // _CPM, i % _CPM, 0)),
            pl.BlockSpec((1, _ROWS, 128), lambda i: (i // _CPM, i % _CPM, 0)),
            pl.BlockSpec(memory_space=pltpu.HBM),
            pl.BlockSpec(memory_space=pltpu.HBM),
        ],
        out_specs=pl.BlockSpec((8, 128), lambda i: (0, 0)),
        out_shape=jax.ShapeDtypeStruct((8, 128), jnp.float32),
        scratch_shapes=[
            pltpu.VMEM((2, _WINR, 128), jnp.float32),
            pltpu.VMEM((2, _WINR, 128), jnp.float32),
            pltpu.SemaphoreType.DMA((2,)),
            pltpu.SemaphoreType.DMA((2,)),
        ],
    )(counts, x3, t3, a1, a2)


def kernel(input, target):
    x3 = input.reshape(_BS, _CPM * _ROWS, 128)
    t3 = target.reshape(_BS, _CPM * _ROWS, 128)
    aug = _dilate(target.reshape(_BS, _H, _W))

    xf = input.reshape(_BS, _L)
    tf = target.reshape(_BS, _L)
    augf = aug.reshape(_BS, _L)
    a1, a2, counts = _compact_sc(xf, tf, augf)

    partials = _loss(counts, x3, t3, a1, a2)
    return jnp.sum(partials).reshape(())


# speedup vs baseline: 6.2486x; 1.0027x over previous
import jax
import jax.numpy as jnp
from jax import lax
from jax.experimental import pallas as pl
from jax.experimental.pallas import tpu as pltpu
from jax.experimental.pallas import tpu_sc as plsc

_BS, _H, _W = 8, 512, 512
_L = _H * _W
_ALPHA = 0.1
_POS_SEED = 5.0
_NEG_SEED = -5.0

_CHUNK = 16384
_CPM = _L // _CHUNK
_ROWS = _CHUNK // 128
_WINR = _ROWS + 2
_EXT = _CHUNK + 2 * 128
_WB = 2048
_WBMAX = (_L + _EXT + _WB - 1) // _WB
_ASTRIDE = _WBMAX * _WB
_AROWS = _ASTRIDE // 128


def _dilate_body(t_ref, o_ref):
    t2 = t_ref[0]
    h = t2
    for s in (1, 2, 3):
        zc = jnp.zeros((_H, s), jnp.float32)
        h = jnp.maximum(h, jnp.concatenate([t2[:, s:], zc], axis=1))
        h = jnp.maximum(h, jnp.concatenate([zc, t2[:, : _W - s]], axis=1))
    v = h
    for s in (1, 2, 3):
        zr = jnp.zeros((s, _W), jnp.float32)
        v = jnp.maximum(v, jnp.concatenate([h[s:, :], zr], axis=0))
        v = jnp.maximum(v, jnp.concatenate([zr, h[: _H - s, :]], axis=0))
    o_ref[0] = (v > 0.0).astype(jnp.float32)


def _dilate(t3):
    return pl.pallas_call(
        _dilate_body,
        grid=(_BS,),
        in_specs=[pl.BlockSpec((1, _H, _W), lambda m: (m, 0, 0))],
        out_specs=pl.BlockSpec((1, _H, _W), lambda m: (m, 0, 0)),
        out_shape=jax.ShapeDtypeStruct((_BS, _H, _W), jnp.float32),
    )(t3)


_NSC = 2
_SUBC = 16
_MAPS_PER_CORE = _BS // _NSC
_CHK = _L // _SUBC
_SUB = 8192
_NSUB = _CHK // _SUB
_VPS = _SUB // 16
_VROWS = _CHK // 128 + 2
_EXTR = (_EXT + 127) // 128
_EXTSLOTS = (_EXTR + _SUBC - 1) // _SUBC


def _sc_body(x_hbm, t_hbm, aug_hbm, a1_hbm, a2_hbm, cnt_hbm,
             xb, tb, gb, vpos, vfp, vneg, idxb, grow, srow, trow,
             crow, tbl, csp, a1sp, a2sp, sem):
    c = lax.axis_index("c")
    s = lax.axis_index("s")
    lane = lax.iota(jnp.int32, 16)
    dump = _L + _EXT

    def popcnt(mask):
        return plsc.cumsum(jnp.where(mask, 1, 0))[15]

    def scatter_local(valref, cnt, base, asp):
        nrows = (cnt + 127) // 128

        def mkrow(j, _):
            for v in range(8):
                p = j * 128 + v * 16 + lane
                iv = jnp.where(p < cnt, base + p, dump + (p & 63))
                idxb[j, pl.ds(v * 16, 16)] = iv
            return 0

        lax.fori_loop(0, nrows, mkrow, 0)

        def fire(j, _):
            pltpu.async_copy(valref.at[pl.ds(j * 128, 128)],
                             asp.at[idxb.at[j]], sem)
            return 0

        lax.fori_loop(0, nrows, fire, 0)

        def drain(j, _):
            pltpu.make_async_copy(valref.at[pl.ds(0, 128)],
                                  asp.at[idxb.at[0]], sem).wait()
            return 0

        lax.fori_loop(0, nrows, drain, 0)

    def extend(asp, n):
        def eloop(jj, _):
            row = s + jj * _SUBC

            @pl.when(row < _EXTR)
            def _():
                for v in range(8):
                    p = n + row * 128 + v * 16 + lane
                    grow[0, pl.ds(v * 16, 16)] = lax.rem(p, n)
                    srow[0, pl.ds(v * 16, 16)] = p
                pltpu.sync_copy(asp.at[grow.at[0]], trow.at[0])
                pltpu.sync_copy(trow.at[0], asp.at[srow.at[0]])
            return 0

        lax.fori_loop(0, _EXTSLOTS, eloop, 0)

    def seed(asp, n, value):
        @pl.when(jnp.logical_and(s == 0, n == 0))
        def _():
            for v in range(8):
                p = v * 16 + lane
                trow[0, pl.ds(v * 16, 16)] = jnp.full((16,), value,
                                                      jnp.float32)
                idxb[0, pl.ds(v * 16, 16)] = jnp.where(
                    p == 0, 0, dump + (p & 63))
            pltpu.sync_copy(trow.at[0], asp.at[idxb.at[0]])

    def writeback(asp, n, ahbm, abase):
        trips = (n + _EXT + _WB - 1) // _WB

        def wloop(jj, _):
            ch = s + jj * _SUBC

            @pl.when(ch < trips)
            def _():
                pltpu.sync_copy(asp.at[pl.ds(ch * _WB, _WB)],
                                ahbm.at[pl.ds(abase + ch * _WB, _WB)])
            return 0

        lax.fori_loop(0, (_WBMAX + _SUBC - 1) // _SUBC, wloop, 0)

    def per_map(mi, _unused):
        m = c * _MAPS_PER_CORE + mi
        gbase = m * _L + s * _CHK
        abase = m * _ASTRIDE

        def sub_loop(sub, carry):
            off = gbase + sub * _SUB
            pltpu.sync_copy(x_hbm.at[pl.ds(off, _SUB)], xb)
            pltpu.sync_copy(t_hbm.at[pl.ds(off, _SUB)], tb)
            pltpu.sync_copy(aug_hbm.at[pl.ds(off, _SUB)], gb)

            def vloop(v, carry2):
                w1, w2, w3 = carry2
                o = v * 16
                xv = xb[pl.ds(o, 16)]
                tv = tb[pl.ds(o, 16)]
                gv = gb[pl.ds(o, 16)]
                pos = tv > 0.0
                neg = gv == 0.0
                fp = jnp.logical_and(xv > 0.0, neg)
                packed = (jnp.where(pos, 1, 0) + jnp.where(fp, 1 << 10, 0)
                          + jnp.where(neg, 1 << 20, 0))
                pk = plsc.cumsum(packed)[15]
                plsc.store_compressed(vpos.at[pl.ds(w1, 16)], xv, mask=pos)
                plsc.store_compressed(vfp.at[pl.ds(w2, 16)], xv, mask=fp)
                plsc.store_compressed(vneg.at[pl.ds(w3, 16)], xv, mask=neg)
                return (w1 + (pk & 0x3FF), w2 + ((pk >> 10) & 0x3FF),
                        w3 + (pk >> 20))

            return lax.fori_loop(0, _VPS, vloop, carry, unroll=8)

        z0 = jnp.int32(0)
        wp1, wp2, wp3 = lax.fori_loop(0, _NSUB, sub_loop, (z0, z0, z0))

        crow[pl.ds(0, 16)] = jnp.where(
            lane == 0, wp1, jnp.where(lane == 1, wp2,
                                      jnp.where(lane == 2, wp3, 0)))
        pltpu.sync_copy(crow, csp.at[c, s])
        plsc.subcore_barrier()
        pltpu.sync_copy(csp.at[c], tbl)

        def offs(j, carry):
            o1, o2, o3, t1, t2, t3 = carry
            rv = tbl[j, pl.ds(0, 16)]
            v1 = rv[0]
            v2 = rv[1]
            v3 = rv[2]
            before = (j < s).astype(jnp.int32)
            return (o1 + before * v1, o2 + before * v2, o3 + before * v3,
                    t1 + v1, t2 + v2, t3 + v3)

        z = jnp.int32(0)
        o1, o2, o3, n1, nf, nn = lax.fori_loop(0, _SUBC, offs,
                                               (z, z, z, z, z, z))
        use_fp = nf > 0
        n2 = jnp.where(use_fp, nf, nn)
        o2c = jnp.where(use_fp, o2, o3)
        c2c = jnp.where(use_fp, wp2, wp3)

        scatter_local(vpos, wp1, o1, a1sp)

        @pl.when(use_fp)
        def _():
            scatter_local(vfp, c2c, o2c, a2sp)

        @pl.when(jnp.logical_not(use_fp))
        def _():
            scatter_local(vneg, c2c, o2c, a2sp)

        seed(a1sp, n1, _POS_SEED)
        seed(a2sp, n2, _NEG_SEED)
        n1e = jnp.maximum(n1, 1)
        n2e = jnp.maximum(n2, 1)
        plsc.subcore_barrier()

        extend(a1sp, n1e)
        extend(a2sp, n2e)
        plsc.subcore_barrier()

        writeback(a1sp, n1e, a1_hbm, abase)
        writeback(a2sp, n2e, a2_hbm, abase)

        @pl.when(s == 0)
        def _():
            crow[pl.ds(0, 16)] = jnp.where(
                lane == 0, n1e, jnp.where(lane == 1, n2e, 0))
            pltpu.sync_copy(crow, cnt_hbm.at[pl.ds(m * 16, 16)])

        plsc.subcore_barrier()
        return 0

    lax.fori_loop(0, _MAPS_PER_CORE, per_map, 0)


def _compact_sc(xf, tf, augf):
    mesh = plsc.VectorSubcoreMesh(core_axis_name="c", subcore_axis_name="s")
    a1, a2, cnt = pl.kernel(
        _sc_body,
        out_type=[
            jax.ShapeDtypeStruct((_BS * _ASTRIDE,), jnp.float32),
            jax.ShapeDtypeStruct((_BS * _ASTRIDE,), jnp.float32),
            jax.ShapeDtypeStruct((_BS * 16,), jnp.int32),
        ],
        mesh=mesh,
        compiler_params=pltpu.CompilerParams(needs_layout_passes=False),
        scratch_types=[
            pltpu.VMEM((_SUB,), jnp.float32),
            pltpu.VMEM((_SUB,), jnp.float32),
            pltpu.VMEM((_SUB,), jnp.float32),
            pltpu.VMEM((_VROWS * 128,), jnp.float32),
            pltpu.VMEM((_VROWS * 128,), jnp.float32),
            pltpu.VMEM((_VROWS * 128,), jnp.float32),
            pltpu.VMEM((_VROWS, 128), jnp.int32),
            pltpu.VMEM((1, 128), jnp.int32),
            pltpu.VMEM((1, 128), jnp.int32),
            pltpu.VMEM((1, 128), jnp.float32),
            pltpu.VMEM((16,), jnp.int32),
            pltpu.VMEM((_SUBC, 16), jnp.int32),
            pltpu.VMEM_SHARED((_NSC, _SUBC, 16), jnp.int32),
            pltpu.VMEM_SHARED((_ASTRIDE,), jnp.float32),
            pltpu.VMEM_SHARED((_ASTRIDE,), jnp.float32),
            pltpu.SemaphoreType.DMA,
        ],
    )(xf.reshape(-1), tf.reshape(-1), augf.reshape(-1))
    return (a1.reshape(_BS, _AROWS, 128), a2.reshape(_BS, _AROWS, 128),
            cnt.reshape(_BS, 16))


def _loss_body(counts_ref, x_ref, t_ref, a1_hbm, a2_hbm, o_ref,
               w1, w2, sem1, sem2):
    i = pl.program_id(0)
    nprog = _BS * _CPM

    def win(idx, slot):
        m = idx // _CPM
        k = idx - m * _CPM
        s1 = lax.rem(k * _CHUNK, counts_ref[m, 0])
        s2 = lax.rem(k * _CHUNK, counts_ref[m, 1])
        c1 = pltpu.make_async_copy(a1_hbm.at[m, pl.ds(s1 // 128, _WINR)],
                                   w1.at[slot], sem1.at[slot])
        c2 = pltpu.make_async_copy(a2_hbm.at[m, pl.ds(s2 // 128, _WINR)],
                                   w2.at[slot], sem2.at[slot])
        return c1, c2, lax.rem(s1, 128), lax.rem(s2, 128)

    slot = lax.rem(i, 2)

    @pl.when(i == 0)
    def _():
        c1, c2, _r1, _r2 = win(i, slot)
        c1.start()
        c2.start()

    @pl.when(i + 1 < nprog)
    def _():
        c1, c2, _r1, _r2 = win(i + 1, lax.rem(i + 1, 2))
        c1.start()
        c2.start()

    c1, c2, r1, r2 = win(i, slot)
    c1.wait()
    c2.wait()

    m = i // _CPM
    lane = lax.broadcasted_iota(jnp.int32, (_ROWS, 128), 1)

    def unshift(w_ref, r):
        wv = w_ref[slot]
        u = pltpu.roll(wv, lax.rem(128 - r, 128), axis=1)
        return jnp.where(lane < 128 - r, u[0:_ROWS], u[1:_ROWS + 1])

    g1 = unshift(w1, r1)
    g2 = unshift(w2, r2)

    x = x_ref[0]
    t = t_ref[0]
    z1 = g1 * x
    z3 = g1 * g2
    e1 = jnp.exp(-jnp.abs(z1))
    e2 = jnp.exp(-jnp.abs(g1))
    flin = (jnp.maximum(z1, 0.0) - z1 * t + jnp.maximum(-g1, 0.0)
            + _ALPHA * jnp.maximum(z3, 0.0))
    flog = jnp.log((1.0 + e1) * (1.0 + e2)) \
        + _ALPHA * jnp.log1p(jnp.exp(-jnp.abs(z3)))
    f = (flin + flog) * (1.0 / (_BS * _L))
    acc = jnp.zeros((8, 128), jnp.float32)
    for rr in range(0, _ROWS, 8):
        acc = acc + f[rr:rr + 8]

    @pl.when(i == 0)
    def _init():
        o_ref[...] = jnp.zeros((8, 128), jnp.float32)

    o_ref[...] += acc


def _loss(counts, x3, t3, a1, a2):
    nprog = _BS * _CPM
    return pl.pallas_call(
        _loss_body,
        grid=(nprog,),
        in_specs=[
            pl.BlockSpec(memory_space=pltpu.SMEM),
            pl.BlockSpec((1, _ROWS, 128), lambda i: (i // _CPM, i % _CPM, 0)),
            pl.BlockSpec((1, _ROWS, 128), lambda i: (i // _CPM, i % _CPM, 0)),
            pl.BlockSpec(memory_space=pltpu.HBM),
            pl.BlockSpec(memory_space=pltpu.HBM),
        ],
        out_specs=pl.BlockSpec((8, 128), lambda i: (0, 0)),
        out_shape=jax.ShapeDtypeStruct((8, 128), jnp.float32),
        scratch_shapes=[
            pltpu.VMEM((2, _WINR, 128), jnp.float32),
            pltpu.VMEM((2, _WINR, 128), jnp.float32),
            pltpu.SemaphoreType.DMA((2,)),
            pltpu.SemaphoreType.DMA((2,)),
        ],
    )(counts, x3, t3, a1, a2)


def kernel(input, target):
    x3 = input.reshape(_BS, _CPM * _ROWS, 128)
    t3 = target.reshape(_BS, _CPM * _ROWS, 128)
    aug = _dilate(target.reshape(_BS, _H, _W))

    xf = input.reshape(_BS, _L)
    tf = target.reshape(_BS, _L)
    augf = aug.reshape(_BS, _L)
    a1, a2, counts = _compact_sc(xf, tf, augf)

    partials = _loss(counts, x3, t3, a1, a2)
    return jnp.sum(partials).reshape(())
